# baseline (speedup 1.0000x reference)
import functools

import jax
import jax.numpy as jnp
from jax import lax
from jax.experimental import pallas as pl
from jax.experimental.pallas import tpu as pltpu
from jax.experimental.pallas import tpu_sc as plsc

N = 50000
E = 1600000
B = 8
NP = 50176
BN = 1024
NBLK = NP // BN
NC = 2
NS = 16
NW = NC * NS
EW = E // NW
K = 200
NIT = EW // K
NSLOT = ((NIT + 2 + 7) // 8) * 8
STRIPE = NP // NS


def _mesh():
    return plsc.VectorSubcoreMesh(core_axis_name="c", subcore_axis_name="s")


def _sc_deg_body(row_hbm, out_hbm, *scr):
    ibuf = list(scr[0:8])
    ones_v = scr[8]
    zbuf = scr[9]
    acc = scr[10]
    isem = list(scr[11:19])
    ssem = list(scr[19:23])
    c = lax.axis_index("c")
    s = lax.axis_index("s")
    w = c * NS + s
    base = w * EW

    def fill(i, _):
        ones_v[pl.ds(i * 16, 16)] = jnp.full((16,), 1.0, jnp.float32)
        return 0

    lax.fori_loop(0, (K + 15) // 16, fill, 0)

    def zfill(i, _):
        zbuf[pl.ds(i * 16, 16)] = jnp.zeros((16,), jnp.float32)
        return 0

    lax.fori_loop(0, STRIPE // 16, zfill, 0)
    for j in range(4):
        off = pl.multiple_of(base + j * K, 8)
        pltpu.async_copy(row_hbm.at[pl.ds(off, K)], ibuf[j], isem[j])
    pltpu.sync_copy(zbuf, acc.at[pl.ds(s * STRIPE, STRIPE)])
    plsc.subcore_barrier()

    def outer(o, _):
        for b8 in range(8):
            i = o * 8 + b8
            b4 = b8 % 4

            @pl.when(i < NIT)
            def _():
                @pl.when(i >= 4)
                def _():
                    pltpu.make_async_copy(
                        ones_v.at[pl.ds(0, K)],
                        acc.at[ibuf[(b8 + 4) % 8]], ssem[b4]).wait()
                off = pl.multiple_of(base + i * K, 8)
                pltpu.make_async_copy(
                    row_hbm.at[pl.ds(off, K)], ibuf[b8], isem[b8]).wait()
                pltpu.async_copy(ones_v.at[pl.ds(0, K)],
                                 acc.at[ibuf[b8]], ssem[b4], add=True)

                @pl.when(i + 4 < NIT)
                def _():
                    off2 = pl.multiple_of(base + (i + 4) * K, 8)
                    pltpu.async_copy(row_hbm.at[pl.ds(off2, K)],
                                     ibuf[(b8 + 4) % 8], isem[(b8 + 4) % 8])
        return 0

    lax.fori_loop(0, NSLOT // 8, outer, 0)
    for q in range(NIT - 4, NIT):
        pltpu.make_async_copy(ones_v.at[pl.ds(0, K)],
                              acc.at[ibuf[q % 8]], ssem[q % 4]).wait()
    plsc.subcore_barrier()
    oof = pl.multiple_of(c * NP + s * STRIPE, 8)
    pltpu.sync_copy(acc.at[pl.ds(s * STRIPE, STRIPE)], zbuf)
    pltpu.sync_copy(zbuf, out_hbm.at[pl.ds(oof, STRIPE)])


def _sc_deg(row):
    f = pl.kernel(
        _sc_deg_body,
        out_type=jax.ShapeDtypeStruct((NC * NP,), jnp.float32),
        mesh=_mesh(),
        scratch_types=(
            [pltpu.VMEM((K,), jnp.int32)] * 8
            + [pltpu.VMEM((((K + 15) // 16) * 16,), jnp.float32)]
            + [pltpu.VMEM((STRIPE,), jnp.float32)]
            + [pltpu.VMEM_SHARED((NP,), jnp.float32)]
            + [pltpu.SemaphoreType.DMA] * 12
        ),
        compiler_params=pltpu.CompilerParams(use_tc_tiling_on_sc=False),
        name="sc_deg",
    )
    return f(row)


ZR = 56
NZ = STRIPE // ZR


def _sc_scat_body(g_hbm, row_hbm, col_hbm, out_hbm, *scr, F, C):
    ridx = list(scr[0:4])
    cidx = list(scr[4:12])
    rows = list(scr[12:16])
    zbuf = scr[16]
    acc = scr[17]
    isr = list(scr[18:22])
    isc = list(scr[22:30])
    gs = list(scr[30:34])
    ss = list(scr[34:38])
    c = lax.axis_index("c")
    s = lax.axis_index("s")
    w = c * NS + s
    base = w * EW

    for cc in range(C):
        gsrc = g_hbm.at[cc]

        def zfill(i, _):
            for c16 in range(F // 16):
                zbuf[i, pl.ds(c16 * 16, 16)] = jnp.zeros((16,), jnp.float32)
            return 0

        lax.fori_loop(0, ZR, zfill, 0)
        for j in range(4):
            off = pl.multiple_of(base + j * K, 8)
            pltpu.async_copy(row_hbm.at[pl.ds(off, K)], ridx[j], isr[j])
            pltpu.async_copy(col_hbm.at[pl.ds(off, K)], cidx[j], isc[j])

        def zcp(z, _):
            zo = pl.multiple_of(s * STRIPE + z * ZR, 8)
            pltpu.sync_copy(zbuf, acc.at[pl.ds(zo, ZR)])
            return 0

        lax.fori_loop(0, NZ, zcp, 0)
        plsc.subcore_barrier()

        def outer(o, _):
            for b8 in range(8):
                i = o * 8 + b8
                b4 = b8 % 4
                bj4 = (b8 - 2) % 4
                bj8 = (b8 - 2) % 8

                @pl.when(i < NIT)
                def _():
                    @pl.when(i >= 4)
                    def _():
                        pltpu.make_async_copy(
                            rows[b4], acc.at[cidx[(b8 + 4) % 8]],
                            ss[b4]).wait()
                    off = pl.multiple_of(base + i * K, 8)
                    pltpu.make_async_copy(
                        row_hbm.at[pl.ds(off, K)], ridx[b4], isr[b4]).wait()
                    pltpu.async_copy(gsrc.at[ridx[b4]], rows[b4], gs[b4])

                    @pl.when(i + 4 < NIT)
                    def _():
                        off2 = pl.multiple_of(base + (i + 4) * K, 8)
                        pltpu.async_copy(col_hbm.at[pl.ds(off2, K)],
                                         cidx[(b8 + 4) % 8],
                                         isc[(b8 + 4) % 8])

                j = i - 2

                @pl.when((j >= 0) & (j < NIT))
                def _():
                    pltpu.make_async_copy(
                        gsrc.at[ridx[bj4]], rows[bj4], gs[bj4]).wait()

                    @pl.when(i + 2 < NIT)
                    def _():
                        off3 = pl.multiple_of(base + (i + 2) * K, 8)
                        pltpu.async_copy(row_hbm.at[pl.ds(off3, K)],
                                         ridx[bj4], isr[bj4])
                    offj = pl.multiple_of(base + j * K, 8)
                    pltpu.make_async_copy(
                        col_hbm.at[pl.ds(offj, K)], cidx[bj8],
                        isc[bj8]).wait()
                    pltpu.async_copy(rows[bj4], acc.at[cidx[bj8]], ss[bj4],
                                     add=True)
            return 0

        lax.fori_loop(0, NSLOT // 8, outer, 0)
        for q in range(NIT - 4, NIT):
            pltpu.make_async_copy(rows[q % 4], acc.at[cidx[q % 8]],
                                  ss[q % 4]).wait()
        plsc.subcore_barrier()

        def fcp(z, _):
            zo = pl.multiple_of(s * STRIPE + z * ZR, 8)
            oof = pl.multiple_of((cc * NC + c) * NP + s * STRIPE + z * ZR, 8)
            pltpu.sync_copy(acc.at[pl.ds(zo, ZR)], zbuf)
            pltpu.sync_copy(zbuf, out_hbm.at[pl.ds(oof, ZR)])
            return 0

        lax.fori_loop(0, NZ, fcp, 0)
        plsc.subcore_barrier()


def _sc_scatter(g, row, col, F, C):
    f = pl.kernel(
        functools.partial(_sc_scat_body, F=F, C=C),
        out_type=jax.ShapeDtypeStruct((C * NC * NP, F), jnp.float32),
        mesh=_mesh(),
        scratch_types=(
            [pltpu.VMEM((K,), jnp.int32)] * 4
            + [pltpu.VMEM((K,), jnp.int32)] * 8
            + [pltpu.VMEM((K, F), jnp.float32)] * 4
            + [pltpu.VMEM((ZR, F), jnp.float32)]
            + [pltpu.VMEM_SHARED((NP, F), jnp.float32)]
            + [pltpu.SemaphoreType.DMA] * 20
        ),
        compiler_params=pltpu.CompilerParams(use_tc_tiling_on_sc=False),
        name="sc_scatter%dx%d" % (F, C),
    )
    return f(g, row, col)


def _tc_prep_body(x_ref, p0_ref, p1_ref, inv_ref, wind_ref, g0_ref):
    deg = p0_ref[...] + p1_ref[...]
    pos = deg > 0.0
    inv = jnp.where(pos, 1.0 / jnp.where(pos, deg, 1.0), 0.0)
    wind = jnp.where(pos, 1.0, 0.0)
    inv_ref[...] = inv
    wind_ref[...] = wind
    g0 = x_ref[...] * inv
    g0_ref[:, 0:3] = g0
    g0_ref[:, 3:16] = jnp.zeros((BN, 13), jnp.float32)


def _tc_prep(x_pad, degp0, degp1):
    return pl.pallas_call(
        _tc_prep_body,
        grid=(NBLK,),
        in_specs=[
            pl.BlockSpec((BN, 3), lambda i: (i, 0)),
            pl.BlockSpec((BN, 1), lambda i: (i, 0)),
            pl.BlockSpec((BN, 1), lambda i: (i, 0)),
        ],
        out_specs=[
            pl.BlockSpec((BN, 1), lambda i: (i, 0)),
            pl.BlockSpec((BN, 1), lambda i: (i, 0)),
            pl.BlockSpec((BN, 16), lambda i: (i, 0)),
        ],
        out_shape=[
            jax.ShapeDtypeStruct((NP, 1), jnp.float32),
            jax.ShapeDtypeStruct((NP, 1), jnp.float32),
            jax.ShapeDtypeStruct((NP, 16), jnp.float32),
        ],
        name="tc_prep",
    )(x_pad, degp0, degp1)


def _tc_layer_body(hin_ref, wind_ref, inv_ref, w0_ref, w1_ref, g_ref, b_ref,
                   *rest, nchunk, cnext, last):
    p_refs = rest[:nchunk]
    if last:
        h_out = rest[nchunk]
    else:
        h_out, y_out, g_out = rest[nchunk:nchunk + 3]
    h = hin_ref[...]
    ax = jnp.concatenate([p[0] + p[1] for p in p_refs], axis=-1)
    ax = ax[:, :h.shape[1]]
    w = wind_ref[...]
    tx1 = (2.0 * w - 1.0) * h - 2.0 * ax
    o = (jnp.dot(h, w0_ref[...], preferred_element_type=jnp.float32)
         + jnp.dot(tx1, w1_ref[...], preferred_element_type=jnp.float32))
    h_out[...] = o
    if not last:
        m = jnp.mean(o, axis=-1, keepdims=True)
        v = jnp.mean((o - m) ** 2, axis=-1, keepdims=True)
        y = (o - m) * lax.rsqrt(v + 1e-5) * g_ref[...] + b_ref[...]
        y = jnp.where(y >= 0.0, y, 0.2 * y)
        y_out[...] = y
        gy = y * inv_ref[...]
        for c in range(cnext):
            g_out[c] = gy[:, c * 32:(c + 1) * 32]


def _tc_layer(hin, wind, inv, w0, w1, g_ln, b_ln, parts, last):
    a, b = w0.shape
    nchunk = len(parts)
    cnext = b // 32
    fch = parts[0].shape[-1]
    in_specs = [
        pl.BlockSpec((BN, a), lambda i: (i, 0)),
        pl.BlockSpec((BN, 1), lambda i: (i, 0)),
        pl.BlockSpec((BN, 1), lambda i: (i, 0)),
        pl.BlockSpec((a, b), lambda i: (0, 0)),
        pl.BlockSpec((a, b), lambda i: (0, 0)),
        pl.BlockSpec((1, b), lambda i: (0, 0)),
        pl.BlockSpec((1, b), lambda i: (0, 0)),
    ] + [pl.BlockSpec((2, BN, fch), lambda i: (0, i, 0))] * nchunk
    out_specs = [pl.BlockSpec((BN, b), lambda i: (i, 0))]
    out_shape = [jax.ShapeDtypeStruct((NP, b), jnp.float32)]
    if not last:
        out_specs += [
            pl.BlockSpec((BN, b), lambda i: (i, 0)),
            pl.BlockSpec((cnext, BN, 32), lambda i: (0, i, 0)),
        ]
        out_shape += [
            jax.ShapeDtypeStruct((NP, b), jnp.float32),
            jax.ShapeDtypeStruct((cnext, NP, 32), jnp.float32),
        ]
    return pl.pallas_call(
        functools.partial(_tc_layer_body, nchunk=nchunk, cnext=cnext,
                          last=last),
        grid=(NBLK,),
        in_specs=in_specs,
        out_specs=out_specs,
        out_shape=out_shape,
        name="tc_layer",
    )(hin, wind, inv, w0, w1, g_ln, b_ln, *parts)


def _tc_pool_body(h1_ref, h2_ref, h3_ref, h4_ref, bat_ref, g_ref, b_ref,
                  w_ref, out_ref, mx_ref, sm_ref, ct_ref):
    i = pl.program_id(0)

    @pl.when(i == 0)
    def _():
        mx_ref[...] = jnp.full((B, 1024), -jnp.inf, jnp.float32)
        sm_ref[...] = jnp.zeros((B, 1024), jnp.float32)
        ct_ref[...] = jnp.zeros((B, 128), jnp.float32)

    cat = jnp.concatenate(
        [h1_ref[...], h2_ref[...], h3_ref[...], h4_ref[...]], axis=-1)
    m = jnp.mean(cat, axis=-1, keepdims=True)
    v = jnp.mean((cat - m) ** 2, axis=-1, keepdims=True)
    y = (cat - m) * lax.rsqrt(v + 1e-5) * g_ref[...] + b_ref[...]
    y = jnp.where(y >= 0.0, y, 0.2 * y)
    j = jnp.dot(y.astype(jnp.bfloat16), w_ref[...],
                preferred_element_type=jnp.float32)
    bat = bat_ref[...]
    bmin = jnp.min(bat)
    bmax = jnp.max(bat)
    for bb in range(B):
        @pl.when((bb >= bmin) & (bb <= bmax))
        def _(bb=bb):
            mask = bat == bb
            jm = jnp.where(mask, j, -jnp.inf)
            mx_ref[bb:bb + 1, :] = jnp.maximum(
                mx_ref[bb:bb + 1, :], jnp.max(jm, axis=0, keepdims=True))
            js = jnp.where(mask, j, 0.0)
            sm_ref[bb:bb + 1, :] = sm_ref[bb:bb + 1, :] + jnp.sum(
                js, axis=0, keepdims=True)
            ct_ref[bb:bb + 1, :] = ct_ref[bb:bb + 1, :] + jnp.sum(
                mask.astype(jnp.float32))

    @pl.when(i == NBLK - 1)
    def _():
        out_ref[:, 0:1024] = mx_ref[...]
        out_ref[:, 1024:2048] = sm_ref[...] / ct_ref[:, 0:1]


def _tc_pool(h1, h2, h3, h4, bat, jk_g, jk_b, jk_W):
    return pl.pallas_call(
        _tc_pool_body,
        grid=(NBLK,),
        in_specs=[
            pl.BlockSpec((BN, 64), lambda i: (i, 0)),
            pl.BlockSpec((BN, 64), lambda i: (i, 0)),
            pl.BlockSpec((BN, 128), lambda i: (i, 0)),
            pl.BlockSpec((BN, 256), lambda i: (i, 0)),
            pl.BlockSpec((BN, 1), lambda i: (i, 0)),
            pl.BlockSpec((1, 512), lambda i: (0, 0)),
            pl.BlockSpec((1, 512), lambda i: (0, 0)),
            pl.BlockSpec((512, 1024), lambda i: (0, 0)),
        ],
        out_specs=pl.BlockSpec((B, 2048), lambda i: (0, 0)),
        out_shape=jax.ShapeDtypeStruct((B, 2048), jnp.float32),
        scratch_shapes=[
            pltpu.VMEM((B, 1024), jnp.float32),
            pltpu.VMEM((B, 1024), jnp.float32),
            pltpu.VMEM((B, 128), jnp.float32),
        ],
        name="tc_pool",
    )(h1, h2, h3, h4, bat, jk_g, jk_b, jk_W.astype(jnp.bfloat16))


def _tc_mlp_body(x_ref, g1_ref, b1_ref, w1_ref, c1_ref,
                 g2_ref, b2_ref, w2_ref, c2_ref,
                 g3_ref, b3_ref, w3_ref, c3_ref, out_ref):
    def lrelu(t):
        return jnp.where(t >= 0.0, t, 0.2 * t)

    o = x_ref[...]
    o = jnp.dot(lrelu(o * g1_ref[...] + b1_ref[...]), w1_ref[...],
                preferred_element_type=jnp.float32) + c1_ref[...]
    o = jnp.dot(lrelu(o * g2_ref[...] + b2_ref[...]), w2_ref[...],
                preferred_element_type=jnp.float32) + c2_ref[...]
    o = jnp.dot(lrelu(o * g3_ref[...] + b3_ref[...]), w3_ref[...],
                preferred_element_type=jnp.float32) + c3_ref[...]
    out_ref[...] = o


def _tc_mlp(pooled, bn1_g, bn1_b, L1w, L1b, bn2_g, bn2_b, L2w, L2b,
            bn3_g, bn3_b, L3w, L3b):
    d1, d2 = L1w.shape
    d3 = L2w.shape[1]
    d4 = L3w.shape[1]
    specs = [pl.BlockSpec(s, lambda i, s=s: tuple(0 for _ in s)) for s in [
        (B, d1), (1, d1), (1, d1), (d1, d2), (1, d2),
        (1, d2), (1, d2), (d2, d3), (1, d3),
        (1, d3), (1, d3), (d3, d4), (1, d4)]]
    return pl.pallas_call(
        _tc_mlp_body,
        grid=(1,),
        in_specs=specs,
        out_specs=pl.BlockSpec((B, d4), lambda i: (0, 0)),
        out_shape=jax.ShapeDtypeStruct((B, d4), jnp.float32),
        name="tc_mlp",
    )(pooled, bn1_g.reshape(1, -1), bn1_b.reshape(1, -1), L1w,
      L1b.reshape(1, -1), bn2_g.reshape(1, -1), bn2_b.reshape(1, -1), L2w,
      L2b.reshape(1, -1), bn3_g.reshape(1, -1), bn3_b.reshape(1, -1), L3w,
      L3b.reshape(1, -1))


def kernel(x, edge_index, batch, c0w0, c0w1, c1w0, c1w1, c2w0, c2w1,
           c3w0, c3w1, g1, b1, g2, b2, g3, b3, jk_g, jk_b, jk_W,
           bn1_g, bn1_b, L1w, L1b, bn2_g, bn2_b, L2w, L2b,
           bn3_g, bn3_b, L3w, L3b):
    row = edge_index[0]
    col = edge_index[1]
    x_pad = jnp.pad(x, ((0, NP - N), (0, 0)))
    bat_pad = jnp.pad(batch, (0, NP - N), constant_values=B).reshape(NP, 1)

    degp = _sc_deg(row).reshape(NC, NP)
    inv, wind, g0 = _tc_prep(x_pad, degp[0].reshape(NP, 1),
                             degp[1].reshape(NP, 1))

    def scat(g, F, C):
        p = _sc_scatter(g, row, col, F, C).reshape(C, NC, NP, F)
        return [p[cc] for cc in range(C)]

    parts = scat(g0.reshape(1, NP, 16), 16, 1)
    h1, y1, gn1 = _tc_layer(x_pad, wind, inv, c0w0, c0w1,
                            g1.reshape(1, -1), b1.reshape(1, -1), parts, False)
    parts = scat(gn1, 32, 2)
    h2, y2, gn2 = _tc_layer(y1, wind, inv, c1w0, c1w1,
                            g2.reshape(1, -1), b2.reshape(1, -1), parts, False)
    parts = scat(gn2, 32, 2)
    h3, y3, gn3 = _tc_layer(y2, wind, inv, c2w0, c2w1,
                            g3.reshape(1, -1), b3.reshape(1, -1), parts, False)
    parts = scat(gn3, 32, 4)
    zb = jnp.zeros((1, c3w0.shape[1]), jnp.float32)
    (h4,) = _tc_layer(y3, wind, inv, c3w0, c3w1, zb, zb, parts, True)

    pooled = _tc_pool(h1, h2, h3, h4, bat_pad, jk_g.reshape(1, -1),
                      jk_b.reshape(1, -1), jk_W)
    return _tc_mlp(pooled, bn1_g, bn1_b, L1w, L1b, bn2_g, bn2_b, L2w, L2b,
                   bn3_g, bn3_b, L3w, L3b)


# speedup vs baseline: 1.2266x; 1.2266x over previous
import functools

import jax
import jax.numpy as jnp
from jax import lax
from jax.experimental import pallas as pl
from jax.experimental.pallas import tpu as pltpu
from jax.experimental.pallas import tpu_sc as plsc

N = 50000
E = 1600000
B = 8
NP = 50176
BN = 1024
NBLK = NP // BN
NC = 2
NS = 16
NW = NC * NS
EW = E // NW
K = 200
NIT = EW // K
NSLOT = ((NIT + 2 + 7) // 8) * 8
STRIPE = NP // NS


def _mesh():
    return plsc.VectorSubcoreMesh(core_axis_name="c", subcore_axis_name="s")


def _sc_deg_body(row_hbm, out_hbm, *scr):
    ibuf = list(scr[0:8])
    ones_v = scr[8]
    zbuf = scr[9]
    acc = scr[10]
    isem = list(scr[11:19])
    ssem = list(scr[19:23])
    c = lax.axis_index("c")
    s = lax.axis_index("s")
    w = c * NS + s
    base = w * EW

    def fill(i, _):
        ones_v[pl.ds(i * 16, 16)] = jnp.full((16,), 1.0, jnp.float32)
        return 0

    lax.fori_loop(0, (K + 15) // 16, fill, 0)

    def zfill(i, _):
        zbuf[pl.ds(i * 16, 16)] = jnp.zeros((16,), jnp.float32)
        return 0

    lax.fori_loop(0, STRIPE // 16, zfill, 0)
    for j in range(4):
        off = pl.multiple_of(base + j * K, 8)
        pltpu.async_copy(row_hbm.at[pl.ds(off, K)], ibuf[j], isem[j])
    pltpu.sync_copy(zbuf, acc.at[pl.ds(s * STRIPE, STRIPE)])
    plsc.subcore_barrier()

    def outer(o, _):
        for b8 in range(8):
            i = o * 8 + b8
            b4 = b8 % 4

            @pl.when(i < NIT)
            def _():
                @pl.when(i >= 4)
                def _():
                    pltpu.make_async_copy(
                        ones_v.at[pl.ds(0, K)],
                        acc.at[ibuf[(b8 + 4) % 8]], ssem[b4]).wait()
                off = pl.multiple_of(base + i * K, 8)
                pltpu.make_async_copy(
                    row_hbm.at[pl.ds(off, K)], ibuf[b8], isem[b8]).wait()
                pltpu.async_copy(ones_v.at[pl.ds(0, K)],
                                 acc.at[ibuf[b8]], ssem[b4], add=True)

                @pl.when(i + 4 < NIT)
                def _():
                    off2 = pl.multiple_of(base + (i + 4) * K, 8)
                    pltpu.async_copy(row_hbm.at[pl.ds(off2, K)],
                                     ibuf[(b8 + 4) % 8], isem[(b8 + 4) % 8])
        return 0

    lax.fori_loop(0, NSLOT // 8, outer, 0)
    for q in range(NIT - 4, NIT):
        pltpu.make_async_copy(ones_v.at[pl.ds(0, K)],
                              acc.at[ibuf[q % 8]], ssem[q % 4]).wait()
    plsc.subcore_barrier()
    oof = pl.multiple_of(c * NP + s * STRIPE, 8)
    pltpu.sync_copy(acc.at[pl.ds(s * STRIPE, STRIPE)], zbuf)
    pltpu.sync_copy(zbuf, out_hbm.at[pl.ds(oof, STRIPE)])


def _sc_deg(row):
    f = pl.kernel(
        _sc_deg_body,
        out_type=jax.ShapeDtypeStruct((NC * NP,), jnp.float32),
        mesh=_mesh(),
        scratch_types=(
            [pltpu.VMEM((K,), jnp.int32)] * 8
            + [pltpu.VMEM((((K + 15) // 16) * 16,), jnp.float32)]
            + [pltpu.VMEM((STRIPE,), jnp.float32)]
            + [pltpu.VMEM_SHARED((NP,), jnp.float32)]
            + [pltpu.SemaphoreType.DMA] * 12
        ),
        compiler_params=pltpu.CompilerParams(use_tc_tiling_on_sc=False),
        name="sc_deg",
    )
    return f(row)


ZR = 56
NZ = STRIPE // ZR


def _sc_scat_body(g_hbm, row_hbm, col_hbm, out_hbm, *scr, F, C):
    ridx = list(scr[0:4])
    cidx = list(scr[4:12])
    rows = list(scr[12:16])
    zbuf = scr[16]
    acc = scr[17]
    isr = list(scr[18:22])
    isc = list(scr[22:30])
    gs = list(scr[30:34])
    ss = list(scr[34:38])
    c = lax.axis_index("c")
    s = lax.axis_index("s")
    w = c * NS + s
    base = w * EW

    for cc in range(C):
        gsrc = g_hbm.at[cc]

        def zfill(i, _):
            for c16 in range(F // 16):
                zbuf[i, pl.ds(c16 * 16, 16)] = jnp.zeros((16,), jnp.float32)
            return 0

        lax.fori_loop(0, ZR, zfill, 0)
        for j in range(4):
            off = pl.multiple_of(base + j * K, 8)
            pltpu.async_copy(row_hbm.at[pl.ds(off, K)], ridx[j], isr[j])
            pltpu.async_copy(col_hbm.at[pl.ds(off, K)], cidx[j], isc[j])

        def zcp(z, _):
            zo = pl.multiple_of(s * STRIPE + z * ZR, 8)
            pltpu.sync_copy(zbuf, acc.at[pl.ds(zo, ZR)])
            return 0

        lax.fori_loop(0, NZ, zcp, 0)
        plsc.subcore_barrier()

        def outer(o, _):
            for b8 in range(8):
                i = o * 8 + b8
                b4 = b8 % 4
                bj4 = (b8 - 2) % 4
                bj8 = (b8 - 2) % 8

                @pl.when(i < NIT)
                def _():
                    @pl.when(i >= 4)
                    def _():
                        pltpu.make_async_copy(
                            rows[b4], acc.at[cidx[(b8 + 4) % 8]],
                            ss[b4]).wait()
                    off = pl.multiple_of(base + i * K, 8)
                    pltpu.make_async_copy(
                        row_hbm.at[pl.ds(off, K)], ridx[b4], isr[b4]).wait()
                    pltpu.async_copy(gsrc.at[ridx[b4]], rows[b4], gs[b4])

                    @pl.when(i + 4 < NIT)
                    def _():
                        off2 = pl.multiple_of(base + (i + 4) * K, 8)
                        pltpu.async_copy(col_hbm.at[pl.ds(off2, K)],
                                         cidx[(b8 + 4) % 8],
                                         isc[(b8 + 4) % 8])

                j = i - 2

                @pl.when((j >= 0) & (j < NIT))
                def _():
                    pltpu.make_async_copy(
                        gsrc.at[ridx[bj4]], rows[bj4], gs[bj4]).wait()

                    @pl.when(i + 2 < NIT)
                    def _():
                        off3 = pl.multiple_of(base + (i + 2) * K, 8)
                        pltpu.async_copy(row_hbm.at[pl.ds(off3, K)],
                                         ridx[bj4], isr[bj4])
                    offj = pl.multiple_of(base + j * K, 8)
                    pltpu.make_async_copy(
                        col_hbm.at[pl.ds(offj, K)], cidx[bj8],
                        isc[bj8]).wait()
                    pltpu.async_copy(rows[bj4], acc.at[cidx[bj8]], ss[bj4],
                                     add=True)
            return 0

        lax.fori_loop(0, NSLOT // 8, outer, 0)
        for q in range(NIT - 4, NIT):
            pltpu.make_async_copy(rows[q % 4], acc.at[cidx[q % 8]],
                                  ss[q % 4]).wait()
        plsc.subcore_barrier()

        def fcp(z, _):
            zo = pl.multiple_of(s * STRIPE + z * ZR, 8)
            oof = pl.multiple_of((cc * NC + c) * NP + s * STRIPE + z * ZR, 8)
            pltpu.sync_copy(acc.at[pl.ds(zo, ZR)], zbuf)
            pltpu.sync_copy(zbuf, out_hbm.at[pl.ds(oof, ZR)])
            return 0

        lax.fori_loop(0, NZ, fcp, 0)
        plsc.subcore_barrier()


def _sc_scatter(g, row, col, F, C):
    f = pl.kernel(
        functools.partial(_sc_scat_body, F=F, C=C),
        out_type=jax.ShapeDtypeStruct((C * NC * NP, F), jnp.float32),
        mesh=_mesh(),
        scratch_types=(
            [pltpu.VMEM((K,), jnp.int32)] * 4
            + [pltpu.VMEM((K,), jnp.int32)] * 8
            + [pltpu.VMEM((K, F), jnp.float32)] * 4
            + [pltpu.VMEM((ZR, F), jnp.float32)]
            + [pltpu.VMEM_SHARED((NP, F), jnp.float32)]
            + [pltpu.SemaphoreType.DMA] * 20
        ),
        compiler_params=pltpu.CompilerParams(use_tc_tiling_on_sc=False),
        name="sc_scatter%dx%d" % (F, C),
    )
    return f(g, row, col)


def _tc_prep_body(x_ref, p0_ref, p1_ref, inv_ref, wind_ref, g0_ref):
    deg = p0_ref[...] + p1_ref[...]
    pos = deg > 0.0
    inv = jnp.where(pos, 1.0 / jnp.where(pos, deg, 1.0), 0.0)
    wind = jnp.where(pos, 1.0, 0.0)
    inv_ref[...] = inv
    wind_ref[...] = wind
    g0 = x_ref[...] * inv
    g0_ref[:, 0:3] = g0
    g0_ref[:, 3:16] = jnp.zeros((BN, 13), jnp.float32)


def _tc_prep(x_pad, degp0, degp1):
    return pl.pallas_call(
        _tc_prep_body,
        grid=(NBLK,),
        in_specs=[
            pl.BlockSpec((BN, 3), lambda i: (i, 0)),
            pl.BlockSpec((BN, 1), lambda i: (i, 0)),
            pl.BlockSpec((BN, 1), lambda i: (i, 0)),
        ],
        out_specs=[
            pl.BlockSpec((BN, 1), lambda i: (i, 0)),
            pl.BlockSpec((BN, 1), lambda i: (i, 0)),
            pl.BlockSpec((BN, 16), lambda i: (i, 0)),
        ],
        out_shape=[
            jax.ShapeDtypeStruct((NP, 1), jnp.float32),
            jax.ShapeDtypeStruct((NP, 1), jnp.float32),
            jax.ShapeDtypeStruct((NP, 16), jnp.float32),
        ],
        name="tc_prep",
    )(x_pad, degp0, degp1)


def _tc_layer_body(hin_ref, wind_ref, inv_ref, w0_ref, w1_ref, g_ref, b_ref,
                   *rest, nchunk, cnext, last):
    p_refs = rest[:nchunk]
    if last:
        h_out = rest[nchunk]
    else:
        h_out, y_out, g_out = rest[nchunk:nchunk + 3]
    h = hin_ref[...]
    ax = jnp.concatenate([p[0] + p[1] for p in p_refs], axis=-1)
    ax = ax[:, :h.shape[1]]
    w = wind_ref[...]
    tx1 = (2.0 * w - 1.0) * h - 2.0 * ax
    o = (jnp.dot(h, w0_ref[...], preferred_element_type=jnp.float32)
         + jnp.dot(tx1, w1_ref[...], preferred_element_type=jnp.float32))
    h_out[...] = o
    if not last:
        m = jnp.mean(o, axis=-1, keepdims=True)
        v = jnp.mean((o - m) ** 2, axis=-1, keepdims=True)
        y = (o - m) * lax.rsqrt(v + 1e-5) * g_ref[...] + b_ref[...]
        y = jnp.where(y >= 0.0, y, 0.2 * y)
        y_out[...] = y
        gy = y * inv_ref[...]
        for c in range(cnext):
            g_out[c] = gy[:, c * 32:(c + 1) * 32]


def _tc_layer(hin, wind, inv, w0, w1, g_ln, b_ln, parts, last):
    a, b = w0.shape
    nchunk = len(parts)
    cnext = b // 32
    fch = parts[0].shape[-1]
    in_specs = [
        pl.BlockSpec((BN, a), lambda i: (i, 0)),
        pl.BlockSpec((BN, 1), lambda i: (i, 0)),
        pl.BlockSpec((BN, 1), lambda i: (i, 0)),
        pl.BlockSpec((a, b), lambda i: (0, 0)),
        pl.BlockSpec((a, b), lambda i: (0, 0)),
        pl.BlockSpec((1, b), lambda i: (0, 0)),
        pl.BlockSpec((1, b), lambda i: (0, 0)),
    ] + [pl.BlockSpec((2, BN, fch), lambda i: (0, i, 0))] * nchunk
    out_specs = [pl.BlockSpec((BN, b), lambda i: (i, 0))]
    out_shape = [jax.ShapeDtypeStruct((NP, b), jnp.float32)]
    if not last:
        out_specs += [
            pl.BlockSpec((BN, b), lambda i: (i, 0)),
            pl.BlockSpec((cnext, BN, 32), lambda i: (0, i, 0)),
        ]
        out_shape += [
            jax.ShapeDtypeStruct((NP, b), jnp.float32),
            jax.ShapeDtypeStruct((cnext, NP, 32), jnp.float32),
        ]
    return pl.pallas_call(
        functools.partial(_tc_layer_body, nchunk=nchunk, cnext=cnext,
                          last=last),
        grid=(NBLK,),
        in_specs=in_specs,
        out_specs=out_specs,
        out_shape=out_shape,
        name="tc_layer",
    )(hin, wind, inv, w0, w1, g_ln, b_ln, *parts)


def _tc_pool_body(h1_ref, h2_ref, h3_ref, h4_ref, bat_ref, g_ref, b_ref,
                  w_ref, out_ref, mx_ref, sm_ref, ct_ref):
    i = pl.program_id(0)

    @pl.when(i == 0)
    def _():
        mx_ref[...] = jnp.full((B, 1024), -jnp.inf, jnp.float32)
        sm_ref[...] = jnp.zeros((B, 1024), jnp.float32)
        ct_ref[...] = jnp.zeros((B, 128), jnp.float32)

    cat = jnp.concatenate(
        [h1_ref[...], h2_ref[...], h3_ref[...], h4_ref[...]], axis=-1)
    m = jnp.mean(cat, axis=-1, keepdims=True)
    v = jnp.mean((cat - m) ** 2, axis=-1, keepdims=True)
    y = (cat - m) * lax.rsqrt(v + 1e-5) * g_ref[...] + b_ref[...]
    y = jnp.where(y >= 0.0, y, 0.2 * y)
    j = jnp.dot(y.astype(jnp.bfloat16), w_ref[...],
                preferred_element_type=jnp.float32)
    bat = bat_ref[...]
    bmin = jnp.min(bat)
    bmax = jnp.max(bat)
    for bb in range(B):
        @pl.when((bb >= bmin) & (bb <= bmax))
        def _(bb=bb):
            mask = bat == bb
            jm = jnp.where(mask, j, -jnp.inf)
            mx_ref[bb:bb + 1, :] = jnp.maximum(
                mx_ref[bb:bb + 1, :], jnp.max(jm, axis=0, keepdims=True))
            js = jnp.where(mask, j, 0.0)
            sm_ref[bb:bb + 1, :] = sm_ref[bb:bb + 1, :] + jnp.sum(
                js, axis=0, keepdims=True)
            ct_ref[bb:bb + 1, :] = ct_ref[bb:bb + 1, :] + jnp.sum(
                mask.astype(jnp.float32))

    @pl.when(i == NBLK - 1)
    def _():
        out_ref[:, 0:1024] = mx_ref[...]
        out_ref[:, 1024:2048] = sm_ref[...] / ct_ref[:, 0:1]


def _tc_pool(h1, h2, h3, h4, bat, jk_g, jk_b, jk_W):
    return pl.pallas_call(
        _tc_pool_body,
        grid=(NBLK,),
        in_specs=[
            pl.BlockSpec((BN, 64), lambda i: (i, 0)),
            pl.BlockSpec((BN, 64), lambda i: (i, 0)),
            pl.BlockSpec((BN, 128), lambda i: (i, 0)),
            pl.BlockSpec((BN, 256), lambda i: (i, 0)),
            pl.BlockSpec((BN, 1), lambda i: (i, 0)),
            pl.BlockSpec((1, 512), lambda i: (0, 0)),
            pl.BlockSpec((1, 512), lambda i: (0, 0)),
            pl.BlockSpec((512, 1024), lambda i: (0, 0)),
        ],
        out_specs=pl.BlockSpec((B, 2048), lambda i: (0, 0)),
        out_shape=jax.ShapeDtypeStruct((B, 2048), jnp.float32),
        scratch_shapes=[
            pltpu.VMEM((B, 1024), jnp.float32),
            pltpu.VMEM((B, 1024), jnp.float32),
            pltpu.VMEM((B, 128), jnp.float32),
        ],
        name="tc_pool",
    )(h1, h2, h3, h4, bat, jk_g, jk_b, jk_W.astype(jnp.bfloat16))


def _tc_mlp_body(x_ref, g1_ref, b1_ref, w1_ref, c1_ref,
                 g2_ref, b2_ref, w2_ref, c2_ref,
                 g3_ref, b3_ref, w3_ref, c3_ref, out_ref):
    def lrelu(t):
        return jnp.where(t >= 0.0, t, 0.2 * t)

    o = x_ref[...]
    o = jnp.dot(lrelu(o * g1_ref[...] + b1_ref[...]), w1_ref[...],
                preferred_element_type=jnp.float32) + c1_ref[...]
    o = jnp.dot(lrelu(o * g2_ref[...] + b2_ref[...]), w2_ref[...],
                preferred_element_type=jnp.float32) + c2_ref[...]
    o = jnp.dot(lrelu(o * g3_ref[...] + b3_ref[...]), w3_ref[...],
                preferred_element_type=jnp.float32) + c3_ref[...]
    out_ref[...] = o


def _tc_mlp(pooled, bn1_g, bn1_b, L1w, L1b, bn2_g, bn2_b, L2w, L2b,
            bn3_g, bn3_b, L3w, L3b):
    d1, d2 = L1w.shape
    d3 = L2w.shape[1]
    d4 = L3w.shape[1]
    specs = [pl.BlockSpec(s, lambda i, s=s: tuple(0 for _ in s)) for s in [
        (B, d1), (1, d1), (1, d1), (d1, d2), (1, d2),
        (1, d2), (1, d2), (d2, d3), (1, d3),
        (1, d3), (1, d3), (d3, d4), (1, d4)]]
    return pl.pallas_call(
        _tc_mlp_body,
        grid=(1,),
        in_specs=specs,
        out_specs=pl.BlockSpec((B, d4), lambda i: (0, 0)),
        out_shape=jax.ShapeDtypeStruct((B, d4), jnp.float32),
        name="tc_mlp",
    )(pooled, bn1_g.reshape(1, -1), bn1_b.reshape(1, -1), L1w,
      L1b.reshape(1, -1), bn2_g.reshape(1, -1), bn2_b.reshape(1, -1), L2w,
      L2b.reshape(1, -1), bn3_g.reshape(1, -1), bn3_b.reshape(1, -1), L3w,
      L3b.reshape(1, -1))


def kernel(x, edge_index, batch, c0w0, c0w1, c1w0, c1w1, c2w0, c2w1,
           c3w0, c3w1, g1, b1, g2, b2, g3, b3, jk_g, jk_b, jk_W,
           bn1_g, bn1_b, L1w, L1b, bn2_g, bn2_b, L2w, L2b,
           bn3_g, bn3_b, L3w, L3b):
    row = edge_index[0]
    col = edge_index[1]
    x_pad = jnp.pad(x, ((0, NP - N), (0, 0)))
    bat_pad = jnp.pad(batch, (0, NP - N), constant_values=B).reshape(NP, 1)

    degp = _sc_deg(row).reshape(NC, NP)
    inv, wind, g0 = _tc_prep(x_pad, degp[0].reshape(NP, 1),
                             degp[1].reshape(NP, 1))

    def scat(g, F, C):
        return [_sc_scatter(g[cc].reshape(1, NP, F), row, col, F, 1)
                .reshape(NC, NP, F) for cc in range(C)]

    parts = scat(g0.reshape(1, NP, 16), 16, 1)
    h1, y1, gn1 = _tc_layer(x_pad, wind, inv, c0w0, c0w1,
                            g1.reshape(1, -1), b1.reshape(1, -1), parts, False)
    parts = scat(gn1, 32, 2)
    h2, y2, gn2 = _tc_layer(y1, wind, inv, c1w0, c1w1,
                            g2.reshape(1, -1), b2.reshape(1, -1), parts, False)
    parts = scat(gn2, 32, 2)
    h3, y3, gn3 = _tc_layer(y2, wind, inv, c2w0, c2w1,
                            g3.reshape(1, -1), b3.reshape(1, -1), parts, False)
    parts = scat(gn3, 32, 4)
    zb = jnp.zeros((1, c3w0.shape[1]), jnp.float32)
    (h4,) = _tc_layer(y3, wind, inv, c3w0, c3w1, zb, zb, parts, True)

    pooled = _tc_pool(h1, h2, h3, h4, bat_pad, jk_g.reshape(1, -1),
                      jk_b.reshape(1, -1), jk_W)
    return _tc_mlp(pooled, bn1_g, bn1_b, L1w, L1b, bn2_g, bn2_b, L2w, L2b,
                   bn3_g, bn3_b, L3w, L3b)


# speedup vs baseline: 1.2445x; 1.0146x over previous
import functools

import jax
import jax.numpy as jnp
from jax import lax
from jax.experimental import pallas as pl
from jax.experimental.pallas import tpu as pltpu
from jax.experimental.pallas import tpu_sc as plsc

N = 50000
E = 1600000
B = 8
NP = 50176
BN = 1024
NBLK = NP // BN
NC = 2
NS = 16
NW = NC * NS
EW = E // NW
K = 200
NIT = EW // K
NSLOT = ((NIT + 2 + 7) // 8) * 8
STRIPE = NP // NS


def _mesh():
    return plsc.VectorSubcoreMesh(core_axis_name="c", subcore_axis_name="s")


def _sc_deg_body(row_hbm, out_hbm, *scr):
    ibuf = list(scr[0:8])
    ones_v = scr[8]
    zbuf = scr[9]
    acc = scr[10]
    isem = list(scr[11:19])
    ssem = list(scr[19:23])
    c = lax.axis_index("c")
    s = lax.axis_index("s")
    w = c * NS + s
    base = w * EW

    def fill(i, _):
        ones_v[pl.ds(i * 16, 16)] = jnp.full((16,), 1.0, jnp.float32)
        return 0

    lax.fori_loop(0, (K + 15) // 16, fill, 0)

    def zfill(i, _):
        zbuf[pl.ds(i * 16, 16)] = jnp.zeros((16,), jnp.float32)
        return 0

    lax.fori_loop(0, STRIPE // 16, zfill, 0)
    for j in range(4):
        off = pl.multiple_of(base + j * K, 8)
        pltpu.async_copy(row_hbm.at[pl.ds(off, K)], ibuf[j], isem[j])
    pltpu.sync_copy(zbuf, acc.at[pl.ds(s * STRIPE, STRIPE)])
    plsc.subcore_barrier()

    def outer(o, _):
        for b8 in range(8):
            i = o * 8 + b8
            b4 = b8 % 4

            @pl.when(i < NIT)
            def _():
                @pl.when(i >= 4)
                def _():
                    pltpu.make_async_copy(
                        ones_v.at[pl.ds(0, K)],
                        acc.at[ibuf[(b8 + 4) % 8]], ssem[b4]).wait()
                off = pl.multiple_of(base + i * K, 8)
                pltpu.make_async_copy(
                    row_hbm.at[pl.ds(off, K)], ibuf[b8], isem[b8]).wait()
                pltpu.async_copy(ones_v.at[pl.ds(0, K)],
                                 acc.at[ibuf[b8]], ssem[b4], add=True)

                @pl.when(i + 4 < NIT)
                def _():
                    off2 = pl.multiple_of(base + (i + 4) * K, 8)
                    pltpu.async_copy(row_hbm.at[pl.ds(off2, K)],
                                     ibuf[(b8 + 4) % 8], isem[(b8 + 4) % 8])
        return 0

    lax.fori_loop(0, NSLOT // 8, outer, 0)
    for q in range(NIT - 4, NIT):
        pltpu.make_async_copy(ones_v.at[pl.ds(0, K)],
                              acc.at[ibuf[q % 8]], ssem[q % 4]).wait()
    plsc.subcore_barrier()
    oof = pl.multiple_of(c * NP + s * STRIPE, 8)
    pltpu.sync_copy(acc.at[pl.ds(s * STRIPE, STRIPE)], zbuf)
    pltpu.sync_copy(zbuf, out_hbm.at[pl.ds(oof, STRIPE)])


def _sc_deg(row):
    f = pl.kernel(
        _sc_deg_body,
        out_type=jax.ShapeDtypeStruct((NC * NP,), jnp.float32),
        mesh=_mesh(),
        scratch_types=(
            [pltpu.VMEM((K,), jnp.int32)] * 8
            + [pltpu.VMEM((((K + 15) // 16) * 16,), jnp.float32)]
            + [pltpu.VMEM((STRIPE,), jnp.float32)]
            + [pltpu.VMEM_SHARED((NP,), jnp.float32)]
            + [pltpu.SemaphoreType.DMA] * 12
        ),
        compiler_params=pltpu.CompilerParams(use_tc_tiling_on_sc=False),
        name="sc_deg",
    )
    return f(row)


ZR = 56
NZ = STRIPE // ZR


def _sc_scat_body(g_hbm, row_hbm, col_hbm, out_hbm, *scr, F, C, CO=0):
    ridx = list(scr[0:4])
    cidx = list(scr[4:12])
    rows = list(scr[12:16])
    zbuf = scr[16]
    acc = scr[17]
    isr = list(scr[18:22])
    isc = list(scr[22:30])
    gs = list(scr[30:34])
    ss = list(scr[34:38])
    c = lax.axis_index("c")
    s = lax.axis_index("s")
    w = c * NS + s
    base = w * EW

    for cc in range(C):
        gsrc = g_hbm.at[cc]

        def zfill(i, _):
            for c16 in range(F // 16):
                zbuf[i, pl.ds(c16 * 16, 16)] = jnp.zeros((16,), jnp.float32)
            return 0

        lax.fori_loop(0, ZR, zfill, 0)
        for j in range(4):
            off = pl.multiple_of(base + j * K, 8)
            pltpu.async_copy(row_hbm.at[pl.ds(off, K)], ridx[j], isr[j])
            pltpu.async_copy(col_hbm.at[pl.ds(off, K)], cidx[j], isc[j])

        def zcp(z, _):
            zo = pl.multiple_of(s * STRIPE + z * ZR, 8)
            pltpu.sync_copy(zbuf, acc.at[pl.ds(zo, ZR)])
            return 0

        lax.fori_loop(0, NZ, zcp, 0)
        plsc.subcore_barrier()

        def outer(o, _):
            for b8 in range(8):
                i = o * 8 + b8
                b4 = b8 % 4
                bj4 = (b8 - 2) % 4
                bj8 = (b8 - 2) % 8

                @pl.when(i < NIT)
                def _():
                    @pl.when(i >= 4)
                    def _():
                        pltpu.make_async_copy(
                            rows[b4], acc.at[cidx[(b8 + 4) % 8]],
                            ss[b4]).wait()
                    off = pl.multiple_of(base + i * K, 8)
                    pltpu.make_async_copy(
                        row_hbm.at[pl.ds(off, K)], ridx[b4], isr[b4]).wait()
                    pltpu.async_copy(gsrc.at[ridx[b4]], rows[b4], gs[b4])

                    @pl.when(i + 4 < NIT)
                    def _():
                        off2 = pl.multiple_of(base + (i + 4) * K, 8)
                        pltpu.async_copy(col_hbm.at[pl.ds(off2, K)],
                                         cidx[(b8 + 4) % 8],
                                         isc[(b8 + 4) % 8])

                j = i - 2

                @pl.when((j >= 0) & (j < NIT))
                def _():
                    pltpu.make_async_copy(
                        gsrc.at[ridx[bj4]], rows[bj4], gs[bj4]).wait()

                    @pl.when(i + 2 < NIT)
                    def _():
                        off3 = pl.multiple_of(base + (i + 2) * K, 8)
                        pltpu.async_copy(row_hbm.at[pl.ds(off3, K)],
                                         ridx[bj4], isr[bj4])
                    offj = pl.multiple_of(base + j * K, 8)
                    pltpu.make_async_copy(
                        col_hbm.at[pl.ds(offj, K)], cidx[bj8],
                        isc[bj8]).wait()
                    pltpu.async_copy(rows[bj4], acc.at[cidx[bj8]], ss[bj4],
                                     add=True)
            return 0

        lax.fori_loop(0, NSLOT // 8, outer, 0)
        for q in range(NIT - 4, NIT):
            pltpu.make_async_copy(rows[q % 4], acc.at[cidx[q % 8]],
                                  ss[q % 4]).wait()
        plsc.subcore_barrier()

        def fcp(z, _):
            zo = pl.multiple_of(s * STRIPE + z * ZR, 8)
            oof = pl.multiple_of((cc * NC + c) * NP + s * STRIPE + z * ZR, 8)
            pltpu.sync_copy(acc.at[pl.ds(zo, ZR)], zbuf)
            pltpu.sync_copy(zbuf, out_hbm.at[pl.ds(oof, ZR)])
            return 0

        lax.fori_loop(0, NZ, fcp, 0)
        plsc.subcore_barrier()


def _sc_scatter(g, row, col, F, C, CO=0):
    f = pl.kernel(
        functools.partial(_sc_scat_body, F=F, C=C, CO=CO),
        out_type=jax.ShapeDtypeStruct((C * NC * NP, F), jnp.float32),
        mesh=_mesh(),
        scratch_types=(
            [pltpu.VMEM((K,), jnp.int32)] * 4
            + [pltpu.VMEM((K,), jnp.int32)] * 8
            + [pltpu.VMEM((K, F), jnp.float32)] * 4
            + [pltpu.VMEM((ZR, F), jnp.float32)]
            + [pltpu.VMEM_SHARED((NP, F), jnp.float32)]
            + [pltpu.SemaphoreType.DMA] * 20
        ),
        compiler_params=pltpu.CompilerParams(use_tc_tiling_on_sc=False),
        name="sc_scatter%dx%d" % (F, C),
    )
    return f(g, row, col)


def _tc_prep_body(x_ref, p0_ref, p1_ref, inv_ref, wind_ref, g0_ref):
    deg = p0_ref[...] + p1_ref[...]
    pos = deg > 0.0
    inv = jnp.where(pos, 1.0 / jnp.where(pos, deg, 1.0), 0.0)
    wind = jnp.where(pos, 1.0, 0.0)
    inv_ref[...] = inv
    wind_ref[...] = wind
    g0 = x_ref[...] * inv
    g0_ref[:, 0:3] = g0
    g0_ref[:, 3:16] = jnp.zeros((BN, 13), jnp.float32)


def _tc_prep(x_pad, degp0, degp1):
    return pl.pallas_call(
        _tc_prep_body,
        grid=(NBLK,),
        in_specs=[
            pl.BlockSpec((BN, 3), lambda i: (i, 0)),
            pl.BlockSpec((BN, 1), lambda i: (i, 0)),
            pl.BlockSpec((BN, 1), lambda i: (i, 0)),
        ],
        out_specs=[
            pl.BlockSpec((BN, 1), lambda i: (i, 0)),
            pl.BlockSpec((BN, 1), lambda i: (i, 0)),
            pl.BlockSpec((BN, 16), lambda i: (i, 0)),
        ],
        out_shape=[
            jax.ShapeDtypeStruct((NP, 1), jnp.float32),
            jax.ShapeDtypeStruct((NP, 1), jnp.float32),
            jax.ShapeDtypeStruct((NP, 16), jnp.float32),
        ],
        name="tc_prep",
    )(x_pad, degp0, degp1)


def _tc_layer_body(hin_ref, wind_ref, inv_ref, w0_ref, w1_ref, g_ref, b_ref,
                   *rest, nchunk, cnext, last):
    p_refs = rest[:nchunk]
    if last:
        h_out = rest[nchunk]
    else:
        h_out, y_out, g_out = rest[nchunk:nchunk + 3]
    h = hin_ref[...]
    ax = jnp.concatenate([p[0] + p[1] for p in p_refs], axis=-1)
    ax = ax[:, :h.shape[1]]
    w = wind_ref[...]
    tx1 = (2.0 * w - 1.0) * h - 2.0 * ax
    o = (jnp.dot(h, w0_ref[...], preferred_element_type=jnp.float32)
         + jnp.dot(tx1, w1_ref[...], preferred_element_type=jnp.float32))
    h_out[...] = o
    if not last:
        m = jnp.mean(o, axis=-1, keepdims=True)
        v = jnp.mean((o - m) ** 2, axis=-1, keepdims=True)
        y = (o - m) * lax.rsqrt(v + 1e-5) * g_ref[...] + b_ref[...]
        y = jnp.where(y >= 0.0, y, 0.2 * y)
        y_out[...] = y
        g_out[...] = y * inv_ref[...]


def _tc_layer(hin, wind, inv, w0, w1, g_ln, b_ln, parts, last):
    a, b = w0.shape
    nchunk = len(parts)
    cnext = b // 32
    fch = parts[0].shape[-1]
    in_specs = [
        pl.BlockSpec((BN, a), lambda i: (i, 0)),
        pl.BlockSpec((BN, 1), lambda i: (i, 0)),
        pl.BlockSpec((BN, 1), lambda i: (i, 0)),
        pl.BlockSpec((a, b), lambda i: (0, 0)),
        pl.BlockSpec((a, b), lambda i: (0, 0)),
        pl.BlockSpec((1, b), lambda i: (0, 0)),
        pl.BlockSpec((1, b), lambda i: (0, 0)),
    ] + [pl.BlockSpec((2, BN, fch), lambda i: (0, i, 0))] * nchunk
    out_specs = [pl.BlockSpec((BN, b), lambda i: (i, 0))]
    out_shape = [jax.ShapeDtypeStruct((NP, b), jnp.float32)]
    if not last:
        out_specs += [
            pl.BlockSpec((BN, b), lambda i: (i, 0)),
            pl.BlockSpec((BN, b), lambda i: (i, 0)),
        ]
        out_shape += [
            jax.ShapeDtypeStruct((NP, b), jnp.float32),
            jax.ShapeDtypeStruct((NP, b), jnp.float32),
        ]
    return pl.pallas_call(
        functools.partial(_tc_layer_body, nchunk=nchunk, cnext=cnext,
                          last=last),
        grid=(NBLK,),
        in_specs=in_specs,
        out_specs=out_specs,
        out_shape=out_shape,
        name="tc_layer",
    )(hin, wind, inv, w0, w1, g_ln, b_ln, *parts)


def _tc_pool_body(h1_ref, h2_ref, h3_ref, h4_ref, bat_ref, g_ref, b_ref,
                  w_ref, out_ref, mx_ref, sm_ref, ct_ref):
    i = pl.program_id(0)

    @pl.when(i == 0)
    def _():
        mx_ref[...] = jnp.full((B, 1024), -jnp.inf, jnp.float32)
        sm_ref[...] = jnp.zeros((B, 1024), jnp.float32)
        ct_ref[...] = jnp.zeros((B, 128), jnp.float32)

    cat = jnp.concatenate(
        [h1_ref[...], h2_ref[...], h3_ref[...], h4_ref[...]], axis=-1)
    m = jnp.mean(cat, axis=-1, keepdims=True)
    v = jnp.mean((cat - m) ** 2, axis=-1, keepdims=True)
    y = (cat - m) * lax.rsqrt(v + 1e-5) * g_ref[...] + b_ref[...]
    y = jnp.where(y >= 0.0, y, 0.2 * y)
    j = jnp.dot(y.astype(jnp.bfloat16), w_ref[...],
                preferred_element_type=jnp.float32)
    bat = bat_ref[...]
    bmin = jnp.min(bat)
    bmax = jnp.max(bat)
    for bb in range(B):
        @pl.when((bb >= bmin) & (bb <= bmax))
        def _(bb=bb):
            mask = bat == bb
            jm = jnp.where(mask, j, -jnp.inf)
            mx_ref[bb:bb + 1, :] = jnp.maximum(
                mx_ref[bb:bb + 1, :], jnp.max(jm, axis=0, keepdims=True))
            js = jnp.where(mask, j, 0.0)
            sm_ref[bb:bb + 1, :] = sm_ref[bb:bb + 1, :] + jnp.sum(
                js, axis=0, keepdims=True)
            ct_ref[bb:bb + 1, :] = ct_ref[bb:bb + 1, :] + jnp.sum(
                mask.astype(jnp.float32))

    @pl.when(i == NBLK - 1)
    def _():
        out_ref[:, 0:1024] = mx_ref[...]
        out_ref[:, 1024:2048] = sm_ref[...] / ct_ref[:, 0:1]


def _tc_pool(h1, h2, h3, h4, bat, jk_g, jk_b, jk_W):
    return pl.pallas_call(
        _tc_pool_body,
        grid=(NBLK,),
        in_specs=[
            pl.BlockSpec((BN, 64), lambda i: (i, 0)),
            pl.BlockSpec((BN, 64), lambda i: (i, 0)),
            pl.BlockSpec((BN, 128), lambda i: (i, 0)),
            pl.BlockSpec((BN, 256), lambda i: (i, 0)),
            pl.BlockSpec((BN, 1), lambda i: (i, 0)),
            pl.BlockSpec((1, 512), lambda i: (0, 0)),
            pl.BlockSpec((1, 512), lambda i: (0, 0)),
            pl.BlockSpec((512, 1024), lambda i: (0, 0)),
        ],
        out_specs=pl.BlockSpec((B, 2048), lambda i: (0, 0)),
        out_shape=jax.ShapeDtypeStruct((B, 2048), jnp.float32),
        scratch_shapes=[
            pltpu.VMEM((B, 1024), jnp.float32),
            pltpu.VMEM((B, 1024), jnp.float32),
            pltpu.VMEM((B, 128), jnp.float32),
        ],
        name="tc_pool",
    )(h1, h2, h3, h4, bat, jk_g, jk_b, jk_W.astype(jnp.bfloat16))


def _tc_mlp_body(x_ref, g1_ref, b1_ref, w1_ref, c1_ref,
                 g2_ref, b2_ref, w2_ref, c2_ref,
                 g3_ref, b3_ref, w3_ref, c3_ref, out_ref):
    def lrelu(t):
        return jnp.where(t >= 0.0, t, 0.2 * t)

    o = x_ref[...]
    o = jnp.dot(lrelu(o * g1_ref[...] + b1_ref[...]), w1_ref[...],
                preferred_element_type=jnp.float32) + c1_ref[...]
    o = jnp.dot(lrelu(o * g2_ref[...] + b2_ref[...]), w2_ref[...],
                preferred_element_type=jnp.float32) + c2_ref[...]
    o = jnp.dot(lrelu(o * g3_ref[...] + b3_ref[...]), w3_ref[...],
                preferred_element_type=jnp.float32) + c3_ref[...]
    out_ref[...] = o


def _tc_mlp(pooled, bn1_g, bn1_b, L1w, L1b, bn2_g, bn2_b, L2w, L2b,
            bn3_g, bn3_b, L3w, L3b):
    d1, d2 = L1w.shape
    d3 = L2w.shape[1]
    d4 = L3w.shape[1]
    specs = [pl.BlockSpec(s, lambda i, s=s: tuple(0 for _ in s)) for s in [
        (B, d1), (1, d1), (1, d1), (d1, d2), (1, d2),
        (1, d2), (1, d2), (d2, d3), (1, d3),
        (1, d3), (1, d3), (d3, d4), (1, d4)]]
    return pl.pallas_call(
        _tc_mlp_body,
        grid=(1,),
        in_specs=specs,
        out_specs=pl.BlockSpec((B, d4), lambda i: (0, 0)),
        out_shape=jax.ShapeDtypeStruct((B, d4), jnp.float32),
        name="tc_mlp",
    )(pooled, bn1_g.reshape(1, -1), bn1_b.reshape(1, -1), L1w,
      L1b.reshape(1, -1), bn2_g.reshape(1, -1), bn2_b.reshape(1, -1), L2w,
      L2b.reshape(1, -1), bn3_g.reshape(1, -1), bn3_b.reshape(1, -1), L3w,
      L3b.reshape(1, -1))


def kernel(x, edge_index, batch, c0w0, c0w1, c1w0, c1w1, c2w0, c2w1,
           c3w0, c3w1, g1, b1, g2, b2, g3, b3, jk_g, jk_b, jk_W,
           bn1_g, bn1_b, L1w, L1b, bn2_g, bn2_b, L2w, L2b,
           bn3_g, bn3_b, L3w, L3b):
    row = edge_index[0]
    col = edge_index[1]
    x_pad = jnp.pad(x, ((0, NP - N), (0, 0)))
    bat_pad = jnp.pad(batch, (0, NP - N), constant_values=B).reshape(NP, 1)

    degp = _sc_deg(row).reshape(NC, NP)
    inv, wind, g0 = _tc_prep(x_pad, degp[0].reshape(NP, 1),
                             degp[1].reshape(NP, 1))

    def scat(g, F, C):
        return [_sc_scatter(g[:, cc * F:(cc + 1) * F].reshape(1, NP, F),
                            row, col, F, 1).reshape(NC, NP, F)
                for cc in range(C)]

    parts = scat(g0, 16, 1)
    h1, y1, gn1 = _tc_layer(x_pad, wind, inv, c0w0, c0w1,
                            g1.reshape(1, -1), b1.reshape(1, -1), parts, False)
    parts = scat(gn1, 32, 2)
    h2, y2, gn2 = _tc_layer(y1, wind, inv, c1w0, c1w1,
                            g2.reshape(1, -1), b2.reshape(1, -1), parts, False)
    parts = scat(gn2, 32, 2)
    h3, y3, gn3 = _tc_layer(y2, wind, inv, c2w0, c2w1,
                            g3.reshape(1, -1), b3.reshape(1, -1), parts, False)
    parts = scat(gn3, 32, 4)
    zb = jnp.zeros((1, c3w0.shape[1]), jnp.float32)
    (h4,) = _tc_layer(y3, wind, inv, c3w0, c3w1, zb, zb, parts, True)

    pooled = _tc_pool(h1, h2, h3, h4, bat_pad, jk_g.reshape(1, -1),
                      jk_b.reshape(1, -1), jk_W)
    return _tc_mlp(pooled, bn1_g, bn1_b, L1w, L1b, bn2_g, bn2_b, L2w, L2b,
                   bn3_g, bn3_b, L3w, L3b)


# speedup vs baseline: 1.3416x; 1.0780x over previous
import functools

import jax
import jax.numpy as jnp
from jax import lax
from jax.experimental import pallas as pl
from jax.experimental.pallas import tpu as pltpu
from jax.experimental.pallas import tpu_sc as plsc

N = 50000
E = 1600000
B = 8
NP = 50176
BN = 1024
NBLK = NP // BN
NC = 2
NS = 16
NW = NC * NS
EW = E // NW
K = 200
NIT = EW // K
NSLOT = ((NIT + 2 + 7) // 8) * 8
STRIPE = NP // NS


def _mesh():
    return plsc.VectorSubcoreMesh(core_axis_name="c", subcore_axis_name="s")


def _sc_deg_body(row_hbm, out_hbm, *scr):
    ibuf = list(scr[0:8])
    ones_v = scr[8]
    zbuf = scr[9]
    acc = scr[10]
    isem = list(scr[11:19])
    ssem = list(scr[19:23])
    c = lax.axis_index("c")
    s = lax.axis_index("s")
    w = c * NS + s
    base = w * EW

    def fill(i, _):
        ones_v[pl.ds(i * 16, 16)] = jnp.full((16,), 1.0, jnp.float32)
        return 0

    lax.fori_loop(0, (K + 15) // 16, fill, 0)

    def zfill(i, _):
        zbuf[pl.ds(i * 16, 16)] = jnp.zeros((16,), jnp.float32)
        return 0

    lax.fori_loop(0, STRIPE // 16, zfill, 0)
    for j in range(4):
        off = pl.multiple_of(base + j * K, 8)
        pltpu.async_copy(row_hbm.at[pl.ds(off, K)], ibuf[j], isem[j])
    pltpu.sync_copy(zbuf, acc.at[pl.ds(s * STRIPE, STRIPE)])
    plsc.subcore_barrier()

    def outer(o, _):
        for b8 in range(8):
            i = o * 8 + b8
            b4 = b8 % 4

            @pl.when(i < NIT)
            def _():
                @pl.when(i >= 4)
                def _():
                    pltpu.make_async_copy(
                        ones_v.at[pl.ds(0, K)],
                        acc.at[ibuf[(b8 + 4) % 8]], ssem[b4]).wait()
                off = pl.multiple_of(base + i * K, 8)
                pltpu.make_async_copy(
                    row_hbm.at[pl.ds(off, K)], ibuf[b8], isem[b8]).wait()
                pltpu.async_copy(ones_v.at[pl.ds(0, K)],
                                 acc.at[ibuf[b8]], ssem[b4], add=True)

                @pl.when(i + 4 < NIT)
                def _():
                    off2 = pl.multiple_of(base + (i + 4) * K, 8)
                    pltpu.async_copy(row_hbm.at[pl.ds(off2, K)],
                                     ibuf[(b8 + 4) % 8], isem[(b8 + 4) % 8])
        return 0

    lax.fori_loop(0, NSLOT // 8, outer, 0)
    for q in range(NIT - 4, NIT):
        pltpu.make_async_copy(ones_v.at[pl.ds(0, K)],
                              acc.at[ibuf[q % 8]], ssem[q % 4]).wait()
    plsc.subcore_barrier()
    oof = pl.multiple_of(c * NP + s * STRIPE, 8)
    pltpu.sync_copy(acc.at[pl.ds(s * STRIPE, STRIPE)], zbuf)
    pltpu.sync_copy(zbuf, out_hbm.at[pl.ds(oof, STRIPE)])


def _sc_deg(row):
    f = pl.kernel(
        _sc_deg_body,
        out_type=jax.ShapeDtypeStruct((NC * NP,), jnp.float32),
        mesh=_mesh(),
        scratch_types=(
            [pltpu.VMEM((K,), jnp.int32)] * 8
            + [pltpu.VMEM((((K + 15) // 16) * 16,), jnp.float32)]
            + [pltpu.VMEM((STRIPE,), jnp.float32)]
            + [pltpu.VMEM_SHARED((NP,), jnp.float32)]
            + [pltpu.SemaphoreType.DMA] * 12
        ),
        compiler_params=pltpu.CompilerParams(use_tc_tiling_on_sc=False),
        name="sc_deg",
    )
    return f(row)


ZR = 56
NZ = STRIPE // ZR


def _sc_scat_body(g_hbm, row_hbm, col_hbm, out_hbm, *scr, F, C, CO=0):
    ridx = list(scr[0:4])
    cidx = list(scr[4:12])
    rows = list(scr[12:16])
    zbuf = scr[16]
    acc = scr[17]
    isr = list(scr[18:22])
    isc = list(scr[22:30])
    gs = list(scr[30:34])
    ss = list(scr[34:38])
    c = lax.axis_index("c")
    s = lax.axis_index("s")
    w = c * NS + s
    base = w * EW

    for cc in range(C):
        gsrc = g_hbm.at[cc]

        def zfill(i, _):
            for c16 in range(F // 16):
                zbuf[i, pl.ds(c16 * 16, 16)] = jnp.zeros((16,), jnp.float32)
            return 0

        lax.fori_loop(0, ZR, zfill, 0)
        for j in range(4):
            off = pl.multiple_of(base + j * K, 8)
            pltpu.async_copy(row_hbm.at[pl.ds(off, K)], ridx[j], isr[j])
            pltpu.async_copy(col_hbm.at[pl.ds(off, K)], cidx[j], isc[j])

        def zcp(z, _):
            zo = pl.multiple_of(s * STRIPE + z * ZR, 8)
            pltpu.sync_copy(zbuf, acc.at[pl.ds(zo, ZR)])
            return 0

        lax.fori_loop(0, NZ, zcp, 0)
        plsc.subcore_barrier()

        def outer(o, _):
            for b8 in range(8):
                i = o * 8 + b8
                b4 = b8 % 4
                bj4 = (b8 - 2) % 4
                bj8 = (b8 - 2) % 8

                @pl.when(i < NIT)
                def _():
                    @pl.when(i >= 4)
                    def _():
                        pltpu.make_async_copy(
                            rows[b4], acc.at[cidx[(b8 + 4) % 8]],
                            ss[b4]).wait()
                    off = pl.multiple_of(base + i * K, 8)
                    pltpu.make_async_copy(
                        row_hbm.at[pl.ds(off, K)], ridx[b4], isr[b4]).wait()
                    pltpu.async_copy(gsrc.at[ridx[b4]], rows[b4], gs[b4])

                    @pl.when(i + 4 < NIT)
                    def _():
                        off2 = pl.multiple_of(base + (i + 4) * K, 8)
                        pltpu.async_copy(col_hbm.at[pl.ds(off2, K)],
                                         cidx[(b8 + 4) % 8],
                                         isc[(b8 + 4) % 8])

                j = i - 2

                @pl.when((j >= 0) & (j < NIT))
                def _():
                    pltpu.make_async_copy(
                        gsrc.at[ridx[bj4]], rows[bj4], gs[bj4]).wait()

                    @pl.when(i + 2 < NIT)
                    def _():
                        off3 = pl.multiple_of(base + (i + 2) * K, 8)
                        pltpu.async_copy(row_hbm.at[pl.ds(off3, K)],
                                         ridx[bj4], isr[bj4])
                    offj = pl.multiple_of(base + j * K, 8)
                    pltpu.make_async_copy(
                        col_hbm.at[pl.ds(offj, K)], cidx[bj8],
                        isc[bj8]).wait()
                    pltpu.async_copy(rows[bj4], acc.at[cidx[bj8]], ss[bj4],
                                     add=True)
            return 0

        lax.fori_loop(0, NSLOT // 8, outer, 0)
        for q in range(NIT - 4, NIT):
            pltpu.make_async_copy(rows[q % 4], acc.at[cidx[q % 8]],
                                  ss[q % 4]).wait()
        plsc.subcore_barrier()

        def fcp(z, _):
            zo = pl.multiple_of(s * STRIPE + z * ZR, 8)
            oof = pl.multiple_of((cc * NC + c) * NP + s * STRIPE + z * ZR, 8)
            pltpu.sync_copy(acc.at[pl.ds(zo, ZR)], zbuf)
            pltpu.sync_copy(zbuf, out_hbm.at[pl.ds(oof, ZR), pl.ds(0, F)])
            return 0

        lax.fori_loop(0, NZ, fcp, 0)
        plsc.subcore_barrier()


def _sc_scatter(g, row, col, F, C, CO=0):
    f = pl.kernel(
        functools.partial(_sc_scat_body, F=F, C=C, CO=CO),
        out_type=jax.ShapeDtypeStruct((C * NC * NP, 128), jnp.float32),
        mesh=_mesh(),
        scratch_types=(
            [pltpu.VMEM((K,), jnp.int32)] * 4
            + [pltpu.VMEM((K,), jnp.int32)] * 8
            + [pltpu.VMEM((K, F), jnp.float32)] * 4
            + [pltpu.VMEM((ZR, F), jnp.float32)]
            + [pltpu.VMEM_SHARED((NP, F), jnp.float32)]
            + [pltpu.SemaphoreType.DMA] * 20
        ),
        compiler_params=pltpu.CompilerParams(use_tc_tiling_on_sc=False),
        name="sc_scatter%dx%d" % (F, C),
    )
    return f(g, row, col)


def _tc_prep_body(x_ref, p0_ref, p1_ref, inv_ref, wind_ref, g0_ref):
    deg = p0_ref[...] + p1_ref[...]
    pos = deg > 0.0
    inv = jnp.where(pos, 1.0 / jnp.where(pos, deg, 1.0), 0.0)
    wind = jnp.where(pos, 1.0, 0.0)
    inv_ref[...] = inv
    wind_ref[...] = wind
    g0 = x_ref[...] * inv
    g0_ref[:, 0:3] = g0
    g0_ref[:, 3:16] = jnp.zeros((BN, 13), jnp.float32)


def _tc_prep(x_pad, degp0, degp1):
    return pl.pallas_call(
        _tc_prep_body,
        grid=(NBLK,),
        in_specs=[
            pl.BlockSpec((BN, 3), lambda i: (i, 0)),
            pl.BlockSpec((BN, 1), lambda i: (i, 0)),
            pl.BlockSpec((BN, 1), lambda i: (i, 0)),
        ],
        out_specs=[
            pl.BlockSpec((BN, 1), lambda i: (i, 0)),
            pl.BlockSpec((BN, 1), lambda i: (i, 0)),
            pl.BlockSpec((BN, 16), lambda i: (i, 0)),
        ],
        out_shape=[
            jax.ShapeDtypeStruct((NP, 1), jnp.float32),
            jax.ShapeDtypeStruct((NP, 1), jnp.float32),
            jax.ShapeDtypeStruct((NP, 16), jnp.float32),
        ],
        name="tc_prep",
    )(x_pad, degp0, degp1)


def _tc_layer_body(hin_ref, wind_ref, inv_ref, w0_ref, w1_ref, g_ref, b_ref,
                   *rest, nchunk, cnext, last, fch):
    p_refs = rest[:2 * nchunk]
    if last:
        h_out = rest[2 * nchunk]
    else:
        h_out, y_out, g_out = rest[2 * nchunk:2 * nchunk + 3]
    h = hin_ref[...]
    ax = jnp.concatenate(
        [p_refs[2 * q][:, :fch] + p_refs[2 * q + 1][:, :fch]
         for q in range(nchunk)], axis=-1)
    ax = ax[:, :h.shape[1]]
    w = wind_ref[...]
    tx1 = (2.0 * w - 1.0) * h - 2.0 * ax
    o = (jnp.dot(h, w0_ref[...], preferred_element_type=jnp.float32)
         + jnp.dot(tx1, w1_ref[...], preferred_element_type=jnp.float32))
    h_out[...] = o
    if not last:
        m = jnp.mean(o, axis=-1, keepdims=True)
        v = jnp.mean((o - m) ** 2, axis=-1, keepdims=True)
        y = (o - m) * lax.rsqrt(v + 1e-5) * g_ref[...] + b_ref[...]
        y = jnp.where(y >= 0.0, y, 0.2 * y)
        y_out[...] = y
        g_out[...] = y * inv_ref[...]


def _tc_layer(hin, wind, inv, w0, w1, g_ln, b_ln, parts, fch, last):
    a, b = w0.shape
    nchunk = len(parts)
    cnext = b // 32
    in_specs = [
        pl.BlockSpec((BN, a), lambda i: (i, 0)),
        pl.BlockSpec((BN, 1), lambda i: (i, 0)),
        pl.BlockSpec((BN, 1), lambda i: (i, 0)),
        pl.BlockSpec((a, b), lambda i: (0, 0)),
        pl.BlockSpec((a, b), lambda i: (0, 0)),
        pl.BlockSpec((1, b), lambda i: (0, 0)),
        pl.BlockSpec((1, b), lambda i: (0, 0)),
    ] + [pl.BlockSpec((BN, 128), lambda i, o=o: (o + i, 0))
         for _ in parts for o in (0, NBLK)]
    out_specs = [pl.BlockSpec((BN, b), lambda i: (i, 0))]
    out_shape = [jax.ShapeDtypeStruct((NP, b), jnp.float32)]
    if not last:
        out_specs += [
            pl.BlockSpec((BN, b), lambda i: (i, 0)),
            pl.BlockSpec((BN, b), lambda i: (i, 0)),
        ]
        out_shape += [
            jax.ShapeDtypeStruct((NP, b), jnp.float32),
            jax.ShapeDtypeStruct((NP, b), jnp.float32),
        ]
    pargs = [p for p in parts for _ in (0, 1)]
    return pl.pallas_call(
        functools.partial(_tc_layer_body, nchunk=nchunk, cnext=cnext,
                          last=last, fch=fch),
        grid=(NBLK,),
        in_specs=in_specs,
        out_specs=out_specs,
        out_shape=out_shape,
        name="tc_layer",
    )(hin, wind, inv, w0, w1, g_ln, b_ln, *pargs)


def _tc_pool_body(h1_ref, h2_ref, h3_ref, h4_ref, bat_ref, g_ref, b_ref,
                  w_ref, out_ref, mx_ref, sm_ref, ct_ref):
    i = pl.program_id(0)

    @pl.when(i == 0)
    def _():
        mx_ref[...] = jnp.full((B, 1024), -jnp.inf, jnp.float32)
        sm_ref[...] = jnp.zeros((B, 1024), jnp.float32)
        ct_ref[...] = jnp.zeros((B, 128), jnp.float32)

    cat = jnp.concatenate(
        [h1_ref[...], h2_ref[...], h3_ref[...], h4_ref[...]], axis=-1)
    m = jnp.mean(cat, axis=-1, keepdims=True)
    v = jnp.mean((cat - m) ** 2, axis=-1, keepdims=True)
    y = (cat - m) * lax.rsqrt(v + 1e-5) * g_ref[...] + b_ref[...]
    y = jnp.where(y >= 0.0, y, 0.2 * y)
    j = jnp.dot(y.astype(jnp.bfloat16), w_ref[...],
                preferred_element_type=jnp.float32)
    bat = bat_ref[...]
    bmin = jnp.min(bat)
    bmax = jnp.max(bat)
    for bb in range(B):
        @pl.when((bb >= bmin) & (bb <= bmax))
        def _(bb=bb):
            mask = bat == bb
            jm = jnp.where(mask, j, -jnp.inf)
            mx_ref[bb:bb + 1, :] = jnp.maximum(
                mx_ref[bb:bb + 1, :], jnp.max(jm, axis=0, keepdims=True))
            js = jnp.where(mask, j, 0.0)
            sm_ref[bb:bb + 1, :] = sm_ref[bb:bb + 1, :] + jnp.sum(
                js, axis=0, keepdims=True)
            ct_ref[bb:bb + 1, :] = ct_ref[bb:bb + 1, :] + jnp.sum(
                mask.astype(jnp.float32))

    @pl.when(i == NBLK - 1)
    def _():
        out_ref[:, 0:1024] = mx_ref[...]
        out_ref[:, 1024:2048] = sm_ref[...] / ct_ref[:, 0:1]


def _tc_pool(h1, h2, h3, h4, bat, jk_g, jk_b, jk_W):
    return pl.pallas_call(
        _tc_pool_body,
        grid=(NBLK,),
        in_specs=[
            pl.BlockSpec((BN, 64), lambda i: (i, 0)),
            pl.BlockSpec((BN, 64), lambda i: (i, 0)),
            pl.BlockSpec((BN, 128), lambda i: (i, 0)),
            pl.BlockSpec((BN, 256), lambda i: (i, 0)),
            pl.BlockSpec((BN, 1), lambda i: (i, 0)),
            pl.BlockSpec((1, 512), lambda i: (0, 0)),
            pl.BlockSpec((1, 512), lambda i: (0, 0)),
            pl.BlockSpec((512, 1024), lambda i: (0, 0)),
        ],
        out_specs=pl.BlockSpec((B, 2048), lambda i: (0, 0)),
        out_shape=jax.ShapeDtypeStruct((B, 2048), jnp.float32),
        scratch_shapes=[
            pltpu.VMEM((B, 1024), jnp.float32),
            pltpu.VMEM((B, 1024), jnp.float32),
            pltpu.VMEM((B, 128), jnp.float32),
        ],
        name="tc_pool",
    )(h1, h2, h3, h4, bat, jk_g, jk_b, jk_W.astype(jnp.bfloat16))


def _tc_mlp_body(x_ref, g1_ref, b1_ref, w1_ref, c1_ref,
                 g2_ref, b2_ref, w2_ref, c2_ref,
                 g3_ref, b3_ref, w3_ref, c3_ref, out_ref):
    def lrelu(t):
        return jnp.where(t >= 0.0, t, 0.2 * t)

    o = x_ref[...]
    o = jnp.dot(lrelu(o * g1_ref[...] + b1_ref[...]), w1_ref[...],
                preferred_element_type=jnp.float32) + c1_ref[...]
    o = jnp.dot(lrelu(o * g2_ref[...] + b2_ref[...]), w2_ref[...],
                preferred_element_type=jnp.float32) + c2_ref[...]
    o = jnp.dot(lrelu(o * g3_ref[...] + b3_ref[...]), w3_ref[...],
                preferred_element_type=jnp.float32) + c3_ref[...]
    out_ref[...] = o


def _tc_mlp(pooled, bn1_g, bn1_b, L1w, L1b, bn2_g, bn2_b, L2w, L2b,
            bn3_g, bn3_b, L3w, L3b):
    d1, d2 = L1w.shape
    d3 = L2w.shape[1]
    d4 = L3w.shape[1]
    specs = [pl.BlockSpec(s, lambda i, s=s: tuple(0 for _ in s)) for s in [
        (B, d1), (1, d1), (1, d1), (d1, d2), (1, d2),
        (1, d2), (1, d2), (d2, d3), (1, d3),
        (1, d3), (1, d3), (d3, d4), (1, d4)]]
    return pl.pallas_call(
        _tc_mlp_body,
        grid=(1,),
        in_specs=specs,
        out_specs=pl.BlockSpec((B, d4), lambda i: (0, 0)),
        out_shape=jax.ShapeDtypeStruct((B, d4), jnp.float32),
        name="tc_mlp",
    )(pooled, bn1_g.reshape(1, -1), bn1_b.reshape(1, -1), L1w,
      L1b.reshape(1, -1), bn2_g.reshape(1, -1), bn2_b.reshape(1, -1), L2w,
      L2b.reshape(1, -1), bn3_g.reshape(1, -1), bn3_b.reshape(1, -1), L3w,
      L3b.reshape(1, -1))


def kernel(x, edge_index, batch, c0w0, c0w1, c1w0, c1w1, c2w0, c2w1,
           c3w0, c3w1, g1, b1, g2, b2, g3, b3, jk_g, jk_b, jk_W,
           bn1_g, bn1_b, L1w, L1b, bn2_g, bn2_b, L2w, L2b,
           bn3_g, bn3_b, L3w, L3b):
    row = edge_index[0]
    col = edge_index[1]
    x_pad = jnp.pad(x, ((0, NP - N), (0, 0)))
    bat_pad = jnp.pad(batch, (0, NP - N), constant_values=B).reshape(NP, 1)

    degp = _sc_deg(row).reshape(NC, NP)
    inv, wind, g0 = _tc_prep(x_pad, degp[0].reshape(NP, 1),
                             degp[1].reshape(NP, 1))

    def scat(g, F, C):
        return [_sc_scatter(g[:, cc * F:(cc + 1) * F].reshape(1, NP, F),
                            row, col, F, 1)
                for cc in range(C)]

    parts = scat(g0, 16, 1)
    h1, y1, gn1 = _tc_layer(x_pad, wind, inv, c0w0, c0w1,
                            g1.reshape(1, -1), b1.reshape(1, -1), parts,
                            16, False)
    parts = scat(gn1, 32, 2)
    h2, y2, gn2 = _tc_layer(y1, wind, inv, c1w0, c1w1,
                            g2.reshape(1, -1), b2.reshape(1, -1), parts,
                            32, False)
    parts = scat(gn2, 32, 2)
    h3, y3, gn3 = _tc_layer(y2, wind, inv, c2w0, c2w1,
                            g3.reshape(1, -1), b3.reshape(1, -1), parts,
                            32, False)
    parts = scat(gn3, 32, 4)
    zb = jnp.zeros((1, c3w0.shape[1]), jnp.float32)
    (h4,) = _tc_layer(y3, wind, inv, c3w0, c3w1, zb, zb, parts, 32, True)

    pooled = _tc_pool(h1, h2, h3, h4, bat_pad, jk_g.reshape(1, -1),
                      jk_b.reshape(1, -1), jk_W)
    return _tc_mlp(pooled, bn1_g, bn1_b, L1w, L1b, bn2_g, bn2_b, L2w, L2b,
                   bn3_g, bn3_b, L3w, L3b)


# speedup vs baseline: 1.3428x; 1.0008x over previous
import functools

import jax
import jax.numpy as jnp
from jax import lax
from jax.experimental import pallas as pl
from jax.experimental.pallas import tpu as pltpu
from jax.experimental.pallas import tpu_sc as plsc

N = 50000
E = 1600000
B = 8
NP = 50176
BN = 1024
NBLK = NP // BN
NC = 2
NS = 16
NW = NC * NS
EW = E // NW
K = 200
NIT = EW // K
NSLOT = ((NIT + 2 + 7) // 8) * 8
STRIPE = NP // NS


def _mesh():
    return plsc.VectorSubcoreMesh(core_axis_name="c", subcore_axis_name="s")


def _sc_deg_body(row_hbm, out_hbm, *scr):
    ibuf = list(scr[0:8])
    ones_v = scr[8]
    zbuf = scr[9]
    acc = scr[10]
    isem = list(scr[11:19])
    ssem = list(scr[19:23])
    c = lax.axis_index("c")
    s = lax.axis_index("s")
    w = c * NS + s
    base = w * EW

    def fill(i, _):
        ones_v[pl.ds(i * 16, 16)] = jnp.full((16,), 1.0, jnp.float32)
        return 0

    lax.fori_loop(0, (K + 15) // 16, fill, 0)

    def zfill(i, _):
        zbuf[pl.ds(i * 16, 16)] = jnp.zeros((16,), jnp.float32)
        return 0

    lax.fori_loop(0, STRIPE // 16, zfill, 0)
    for j in range(4):
        off = pl.multiple_of(base + j * K, 8)
        pltpu.async_copy(row_hbm.at[pl.ds(off, K)], ibuf[j], isem[j])
    pltpu.sync_copy(zbuf, acc.at[pl.ds(s * STRIPE, STRIPE)])
    plsc.subcore_barrier()

    def outer(o, _):
        for b8 in range(8):
            i = o * 8 + b8
            b4 = b8 % 4

            @pl.when(i < NIT)
            def _():
                @pl.when(i >= 4)
                def _():
                    pltpu.make_async_copy(
                        ones_v.at[pl.ds(0, K)],
                        acc.at[ibuf[(b8 + 4) % 8]], ssem[b4]).wait()
                off = pl.multiple_of(base + i * K, 8)
                pltpu.make_async_copy(
                    row_hbm.at[pl.ds(off, K)], ibuf[b8], isem[b8]).wait()
                pltpu.async_copy(ones_v.at[pl.ds(0, K)],
                                 acc.at[ibuf[b8]], ssem[b4], add=True)

                @pl.when(i + 4 < NIT)
                def _():
                    off2 = pl.multiple_of(base + (i + 4) * K, 8)
                    pltpu.async_copy(row_hbm.at[pl.ds(off2, K)],
                                     ibuf[(b8 + 4) % 8], isem[(b8 + 4) % 8])
        return 0

    lax.fori_loop(0, NSLOT // 8, outer, 0)
    for q in range(NIT - 4, NIT):
        pltpu.make_async_copy(ones_v.at[pl.ds(0, K)],
                              acc.at[ibuf[q % 8]], ssem[q % 4]).wait()
    plsc.subcore_barrier()
    oof = pl.multiple_of(c * NP + s * STRIPE, 8)
    pltpu.sync_copy(acc.at[pl.ds(s * STRIPE, STRIPE)], zbuf)
    pltpu.sync_copy(zbuf, out_hbm.at[pl.ds(oof, STRIPE)])


def _sc_deg(row):
    f = pl.kernel(
        _sc_deg_body,
        out_type=jax.ShapeDtypeStruct((NC * NP,), jnp.float32),
        mesh=_mesh(),
        scratch_types=(
            [pltpu.VMEM((K,), jnp.int32)] * 8
            + [pltpu.VMEM((((K + 15) // 16) * 16,), jnp.float32)]
            + [pltpu.VMEM((STRIPE,), jnp.float32)]
            + [pltpu.VMEM_SHARED((NP,), jnp.float32)]
            + [pltpu.SemaphoreType.DMA] * 12
        ),
        compiler_params=pltpu.CompilerParams(use_tc_tiling_on_sc=False),
        name="sc_deg",
    )
    return f(row)


ZR = 56
NZ = STRIPE // ZR


def _sc_scat_body(g_hbm, row_hbm, col_hbm, out_hbm, *scr, F, C):
    ridx = list(scr[0:4])
    cidx = list(scr[4:12])
    rows = list(scr[12:16])
    zbuf = scr[16]
    acc = scr[17]
    isr = list(scr[18:22])
    isc = list(scr[22:30])
    gs = list(scr[30:34])
    ss = list(scr[34:38])
    c = lax.axis_index("c")
    s = lax.axis_index("s")
    w = c * NS + s
    base = w * EW

    for cc in range(C):
        gsrc = g_hbm.at[cc]

        def zfill(i, _):
            for c16 in range(F // 16):
                zbuf[i, pl.ds(c16 * 16, 16)] = jnp.zeros((16,), jnp.float32)
            return 0

        lax.fori_loop(0, ZR, zfill, 0)
        for j in range(4):
            off = pl.multiple_of(base + j * K, 8)
            pltpu.async_copy(row_hbm.at[pl.ds(off, K)], ridx[j], isr[j])
            pltpu.async_copy(col_hbm.at[pl.ds(off, K)], cidx[j], isc[j])

        def zcp(z, _):
            zo = pl.multiple_of(s * STRIPE + z * ZR, 8)
            pltpu.sync_copy(zbuf, acc.at[pl.ds(zo, ZR)])
            return 0

        lax.fori_loop(0, NZ, zcp, 0)
        plsc.subcore_barrier()

        def outer(o, _):
            for b8 in range(8):
                i = o * 8 + b8
                b4 = b8 % 4
                bj4 = (b8 - 2) % 4
                bj8 = (b8 - 2) % 8

                @pl.when(i < NIT)
                def _():
                    @pl.when(i >= 4)
                    def _():
                        pltpu.make_async_copy(
                            rows[b4], acc.at[cidx[(b8 + 4) % 8]],
                            ss[b4]).wait()
                    off = pl.multiple_of(base + i * K, 8)
                    pltpu.make_async_copy(
                        row_hbm.at[pl.ds(off, K)], ridx[b4], isr[b4]).wait()
                    pltpu.async_copy(gsrc.at[ridx[b4]], rows[b4], gs[b4])

                    @pl.when(i + 4 < NIT)
                    def _():
                        off2 = pl.multiple_of(base + (i + 4) * K, 8)
                        pltpu.async_copy(col_hbm.at[pl.ds(off2, K)],
                                         cidx[(b8 + 4) % 8],
                                         isc[(b8 + 4) % 8])

                j = i - 2

                @pl.when((j >= 0) & (j < NIT))
                def _():
                    pltpu.make_async_copy(
                        gsrc.at[ridx[bj4]], rows[bj4], gs[bj4]).wait()

                    @pl.when(i + 2 < NIT)
                    def _():
                        off3 = pl.multiple_of(base + (i + 2) * K, 8)
                        pltpu.async_copy(row_hbm.at[pl.ds(off3, K)],
                                         ridx[bj4], isr[bj4])
                    offj = pl.multiple_of(base + j * K, 8)
                    pltpu.make_async_copy(
                        col_hbm.at[pl.ds(offj, K)], cidx[bj8],
                        isc[bj8]).wait()
                    pltpu.async_copy(rows[bj4], acc.at[cidx[bj8]], ss[bj4],
                                     add=True)
            return 0

        lax.fori_loop(0, NSLOT // 8, outer, 0)
        for q in range(NIT - 4, NIT):
            pltpu.make_async_copy(rows[q % 4], acc.at[cidx[q % 8]],
                                  ss[q % 4]).wait()
        plsc.subcore_barrier()

        def fcp(z, _):
            zo = pl.multiple_of(s * STRIPE + z * ZR, 8)
            oof = pl.multiple_of((cc * NC + c) * NP + s * STRIPE + z * ZR, 8)
            pltpu.sync_copy(acc.at[pl.ds(zo, ZR)], zbuf)
            pltpu.sync_copy(zbuf, out_hbm.at[pl.ds(oof, ZR), pl.ds(0, F)])
            return 0

        lax.fori_loop(0, NZ, fcp, 0)
        plsc.subcore_barrier()


def _sc_scatter(g, row, col, F, C):
    f = pl.kernel(
        functools.partial(_sc_scat_body, F=F, C=C),
        out_type=jax.ShapeDtypeStruct((C * NC * NP, 128), jnp.float32),
        mesh=_mesh(),
        scratch_types=(
            [pltpu.VMEM((K,), jnp.int32)] * 4
            + [pltpu.VMEM((K,), jnp.int32)] * 8
            + [pltpu.VMEM((K, F), jnp.float32)] * 4
            + [pltpu.VMEM((ZR, F), jnp.float32)]
            + [pltpu.VMEM_SHARED((NP, F), jnp.float32)]
            + [pltpu.SemaphoreType.DMA] * 20
        ),
        compiler_params=pltpu.CompilerParams(use_tc_tiling_on_sc=False),
        name="sc_scatter%dx%d" % (F, C),
    )
    return f(g, row, col)


def _tc_prep_body(x_ref, p0_ref, p1_ref, inv_ref, wind_ref, g0_ref):
    deg = p0_ref[...] + p1_ref[...]
    pos = deg > 0.0
    inv = jnp.where(pos, 1.0 / jnp.where(pos, deg, 1.0), 0.0)
    wind = jnp.where(pos, 1.0, 0.0)
    inv_ref[...] = inv
    wind_ref[...] = wind
    g0 = x_ref[...] * inv
    g0_ref[:, 0:3] = g0
    g0_ref[:, 3:16] = jnp.zeros((BN, 13), jnp.float32)


def _tc_prep(x_pad, degp0, degp1):
    return pl.pallas_call(
        _tc_prep_body,
        grid=(NBLK,),
        in_specs=[
            pl.BlockSpec((BN, 3), lambda i: (i, 0)),
            pl.BlockSpec((BN, 1), lambda i: (i, 0)),
            pl.BlockSpec((BN, 1), lambda i: (i, 0)),
        ],
        out_specs=[
            pl.BlockSpec((BN, 1), lambda i: (i, 0)),
            pl.BlockSpec((BN, 1), lambda i: (i, 0)),
            pl.BlockSpec((BN, 16), lambda i: (i, 0)),
        ],
        out_shape=[
            jax.ShapeDtypeStruct((NP, 1), jnp.float32),
            jax.ShapeDtypeStruct((NP, 1), jnp.float32),
            jax.ShapeDtypeStruct((NP, 16), jnp.float32),
        ],
        name="tc_prep",
    )(x_pad, degp0, degp1)


def _tc_layer_body(hin_ref, wind_ref, inv_ref, w0_ref, w1_ref, g_ref, b_ref,
                   *rest, nchunk, cnext, last, fch):
    p_refs = rest[:2 * nchunk]
    if last:
        h_out = rest[2 * nchunk]
    else:
        h_out, y_out, g_out = rest[2 * nchunk:2 * nchunk + 3]
    h = hin_ref[...]
    ax = jnp.concatenate(
        [p_refs[2 * q][:, :fch] + p_refs[2 * q + 1][:, :fch]
         for q in range(nchunk)], axis=-1)
    ax = ax[:, :h.shape[1]]
    w = wind_ref[...]
    tx1 = (2.0 * w - 1.0) * h - 2.0 * ax
    o = (jnp.dot(h, w0_ref[...], preferred_element_type=jnp.float32)
         + jnp.dot(tx1, w1_ref[...], preferred_element_type=jnp.float32))
    h_out[...] = o
    if not last:
        m = jnp.mean(o, axis=-1, keepdims=True)
        v = jnp.mean((o - m) ** 2, axis=-1, keepdims=True)
        y = (o - m) * lax.rsqrt(v + 1e-5) * g_ref[...] + b_ref[...]
        y = jnp.where(y >= 0.0, y, 0.2 * y)
        y_out[...] = y
        g_out[...] = y * inv_ref[...]


def _tc_layer(hin, wind, inv, w0, w1, g_ln, b_ln, parts, fch, last):
    a, b = w0.shape
    nchunk = len(parts)
    cnext = b // 32
    in_specs = [
        pl.BlockSpec((BN, a), lambda i: (i, 0)),
        pl.BlockSpec((BN, 1), lambda i: (i, 0)),
        pl.BlockSpec((BN, 1), lambda i: (i, 0)),
        pl.BlockSpec((a, b), lambda i: (0, 0)),
        pl.BlockSpec((a, b), lambda i: (0, 0)),
        pl.BlockSpec((1, b), lambda i: (0, 0)),
        pl.BlockSpec((1, b), lambda i: (0, 0)),
    ] + [pl.BlockSpec((BN, 128), lambda i, o=o: (o + i, 0))
         for _ in parts for o in (0, NBLK)]
    out_specs = [pl.BlockSpec((BN, b), lambda i: (i, 0))]
    out_shape = [jax.ShapeDtypeStruct((NP, b), jnp.float32)]
    if not last:
        out_specs += [
            pl.BlockSpec((BN, b), lambda i: (i, 0)),
            pl.BlockSpec((BN, b), lambda i: (i, 0)),
        ]
        out_shape += [
            jax.ShapeDtypeStruct((NP, b), jnp.float32),
            jax.ShapeDtypeStruct((NP, b), jnp.float32),
        ]
    pargs = [p for p in parts for _ in (0, 1)]
    return pl.pallas_call(
        functools.partial(_tc_layer_body, nchunk=nchunk, cnext=cnext,
                          last=last, fch=fch),
        grid=(NBLK,),
        in_specs=in_specs,
        out_specs=out_specs,
        out_shape=out_shape,
        name="tc_layer",
    )(hin, wind, inv, w0, w1, g_ln, b_ln, *pargs)


def _tc_pool_body(h1_ref, h2_ref, h3_ref, h4_ref, bat_ref, g_ref, b_ref,
                  w_ref, out_ref, mx_ref, sm_ref, ct_ref):
    i = pl.program_id(0)

    @pl.when(i == 0)
    def _():
        mx_ref[...] = jnp.full((B, 1024), -jnp.inf, jnp.float32)
        sm_ref[...] = jnp.zeros((B, 1024), jnp.float32)
        ct_ref[...] = jnp.zeros((B, 128), jnp.float32)

    cat = jnp.concatenate(
        [h1_ref[...], h2_ref[...], h3_ref[...], h4_ref[...]], axis=-1)
    m = jnp.mean(cat, axis=-1, keepdims=True)
    v = jnp.mean((cat - m) ** 2, axis=-1, keepdims=True)
    y = (cat - m) * lax.rsqrt(v + 1e-5) * g_ref[...] + b_ref[...]
    y = jnp.where(y >= 0.0, y, 0.2 * y)
    j = jnp.dot(y.astype(jnp.bfloat16), w_ref[...],
                preferred_element_type=jnp.float32)
    bat = bat_ref[...]
    bmin = jnp.min(bat)
    bmax = jnp.max(bat)
    for bb in range(B):
        @pl.when((bb >= bmin) & (bb <= bmax))
        def _(bb=bb):
            mask = bat == bb
            jm = jnp.where(mask, j, -jnp.inf)
            mx_ref[bb:bb + 1, :] = jnp.maximum(
                mx_ref[bb:bb + 1, :], jnp.max(jm, axis=0, keepdims=True))
            js = jnp.where(mask, j, 0.0)
            sm_ref[bb:bb + 1, :] = sm_ref[bb:bb + 1, :] + jnp.sum(
                js, axis=0, keepdims=True)
            ct_ref[bb:bb + 1, :] = ct_ref[bb:bb + 1, :] + jnp.sum(
                mask.astype(jnp.float32))

    @pl.when(i == NBLK - 1)
    def _():
        out_ref[:, 0:1024] = mx_ref[...]
        out_ref[:, 1024:2048] = sm_ref[...] / ct_ref[:, 0:1]


def _tc_pool(h1, h2, h3, h4, bat, jk_g, jk_b, jk_W):
    return pl.pallas_call(
        _tc_pool_body,
        grid=(NBLK,),
        in_specs=[
            pl.BlockSpec((BN, 64), lambda i: (i, 0)),
            pl.BlockSpec((BN, 64), lambda i: (i, 0)),
            pl.BlockSpec((BN, 128), lambda i: (i, 0)),
            pl.BlockSpec((BN, 256), lambda i: (i, 0)),
            pl.BlockSpec((BN, 1), lambda i: (i, 0)),
            pl.BlockSpec((1, 512), lambda i: (0, 0)),
            pl.BlockSpec((1, 512), lambda i: (0, 0)),
            pl.BlockSpec((512, 1024), lambda i: (0, 0)),
        ],
        out_specs=pl.BlockSpec((B, 2048), lambda i: (0, 0)),
        out_shape=jax.ShapeDtypeStruct((B, 2048), jnp.float32),
        scratch_shapes=[
            pltpu.VMEM((B, 1024), jnp.float32),
            pltpu.VMEM((B, 1024), jnp.float32),
            pltpu.VMEM((B, 128), jnp.float32),
        ],
        name="tc_pool",
    )(h1, h2, h3, h4, bat, jk_g, jk_b, jk_W.astype(jnp.bfloat16))


def _tc_mlp_body(x_ref, g1_ref, b1_ref, w1_ref, c1_ref,
                 g2_ref, b2_ref, w2_ref, c2_ref,
                 g3_ref, b3_ref, w3_ref, c3_ref, out_ref):
    def lrelu(t):
        return jnp.where(t >= 0.0, t, 0.2 * t)

    o = x_ref[...]
    o = jnp.dot(lrelu(o * g1_ref[...] + b1_ref[...]), w1_ref[...],
                preferred_element_type=jnp.float32) + c1_ref[...]
    o = jnp.dot(lrelu(o * g2_ref[...] + b2_ref[...]), w2_ref[...],
                preferred_element_type=jnp.float32) + c2_ref[...]
    o = jnp.dot(lrelu(o * g3_ref[...] + b3_ref[...]), w3_ref[...],
                preferred_element_type=jnp.float32) + c3_ref[...]
    out_ref[...] = o


def _tc_mlp(pooled, bn1_g, bn1_b, L1w, L1b, bn2_g, bn2_b, L2w, L2b,
            bn3_g, bn3_b, L3w, L3b):
    d1, d2 = L1w.shape
    d3 = L2w.shape[1]
    d4 = L3w.shape[1]
    specs = [pl.BlockSpec(s, lambda i, s=s: tuple(0 for _ in s)) for s in [
        (B, d1), (1, d1), (1, d1), (d1, d2), (1, d2),
        (1, d2), (1, d2), (d2, d3), (1, d3),
        (1, d3), (1, d3), (d3, d4), (1, d4)]]
    return pl.pallas_call(
        _tc_mlp_body,
        grid=(1,),
        in_specs=specs,
        out_specs=pl.BlockSpec((B, d4), lambda i: (0, 0)),
        out_shape=jax.ShapeDtypeStruct((B, d4), jnp.float32),
        name="tc_mlp",
    )(pooled, bn1_g.reshape(1, -1), bn1_b.reshape(1, -1), L1w,
      L1b.reshape(1, -1), bn2_g.reshape(1, -1), bn2_b.reshape(1, -1), L2w,
      L2b.reshape(1, -1), bn3_g.reshape(1, -1), bn3_b.reshape(1, -1), L3w,
      L3b.reshape(1, -1))


def kernel(x, edge_index, batch, c0w0, c0w1, c1w0, c1w1, c2w0, c2w1,
           c3w0, c3w1, g1, b1, g2, b2, g3, b3, jk_g, jk_b, jk_W,
           bn1_g, bn1_b, L1w, L1b, bn2_g, bn2_b, L2w, L2b,
           bn3_g, bn3_b, L3w, L3b):
    row = edge_index[0]
    col = edge_index[1]
    x_pad = jnp.pad(x, ((0, NP - N), (0, 0)))
    bat_pad = jnp.pad(batch, (0, NP - N), constant_values=B).reshape(NP, 1)

    degp = _sc_deg(row).reshape(NC, NP)
    inv, wind, g0 = _tc_prep(x_pad, degp[0].reshape(NP, 1),
                             degp[1].reshape(NP, 1))

    def scat(g, F, C):
        return [_sc_scatter(g[:, cc * F:(cc + 1) * F].reshape(1, NP, F),
                            row, col, F, 1)
                for cc in range(C)]

    parts = scat(g0, 16, 1)
    h1, y1, gn1 = _tc_layer(x_pad, wind, inv, c0w0, c0w1,
                            g1.reshape(1, -1), b1.reshape(1, -1), parts,
                            16, False)
    parts = scat(gn1, 32, 2)
    h2, y2, gn2 = _tc_layer(y1, wind, inv, c1w0, c1w1,
                            g2.reshape(1, -1), b2.reshape(1, -1), parts,
                            32, False)
    parts = scat(gn2, 32, 2)
    h3, y3, gn3 = _tc_layer(y2, wind, inv, c2w0, c2w1,
                            g3.reshape(1, -1), b3.reshape(1, -1), parts,
                            32, False)
    parts = scat(gn3, 32, 4)
    zb = jnp.zeros((1, c3w0.shape[1]), jnp.float32)
    (h4,) = _tc_layer(y3, wind, inv, c3w0, c3w1, zb, zb, parts, 32, True)

    pooled = _tc_pool(h1, h2, h3, h4, bat_pad, jk_g.reshape(1, -1),
                      jk_b.reshape(1, -1), jk_W)
    return _tc_mlp(pooled, bn1_g, bn1_b, L1w, L1b, bn2_g, bn2_b, L2w, L2b,
                   bn3_g, bn3_b, L3w, L3b)
